# Initial kernel scaffold; baseline (speedup 1.0000x reference)
import jax
import jax.numpy as jnp
from jax.experimental import pallas as pl


def kernel(x, edge_index, W1, a_src1, a_dst1, b1, W2, a_src2, a_dst2, b2):
    raise NotImplementedError("write your pallas kernel here")


# speedup vs baseline: 20.1771x; 20.1771x over previous
import functools

import jax
import jax.numpy as jnp
from jax import lax
import numpy as np
from jax.experimental import pallas as pl
from jax.experimental.pallas import tpu as pltpu
from jax.experimental.pallas import tpu_sc as plsc

_N = 10000
_E = 320000
_D = 128
_HID = 64
_HEADS = 8
_NCLS = 40

_NP = 10240
_NPT = _NP // 16
_K = 128
_K1 = 64
_EE = _E + _N
_EEP = 331776
_P1T = _EEP // 16
_P2T = _EEP // 32
_N1 = _P1T // _K
_N2 = _P2T // _K
_N11 = _P1T // _K1
_N21 = _P2T // _K1
_C2 = 48


def _leaky(x):
    return jnp.where(x >= 0.0, x, 0.2 * x)


_GDN = lax.GatherDimensionNumbers(offset_dims=(), collapsed_slice_dims=(0,),
                                  start_index_map=(0,))


def _vbcast(v, lane):
    idx = jnp.full((16, 1), lane, jnp.int32)
    return lax.gather(v, idx, _GDN, (1,),
                      mode=lax.GatherScatterMode.PROMISE_IN_BOUNDS)


def _mm1_body(x_ref, w_ref, asw_ref, adw_ref, h_ref, as_ref, ad_ref):
    h = jnp.dot(x_ref[...], w_ref[...], preferred_element_type=jnp.float32)
    h_ref[...] = h
    as_ref[...] = jnp.dot(h, asw_ref[...], preferred_element_type=jnp.float32)
    ad_ref[...] = jnp.dot(h, adw_ref[...], preferred_element_type=jnp.float32)


def _mm1(xp, W1, asw, adw, bm=1024):
    return pl.pallas_call(
        _mm1_body,
        grid=(_NP // bm,),
        in_specs=[pl.BlockSpec((bm, _D), lambda i: (i, 0)),
                  pl.BlockSpec((_D, 512), lambda i: (0, 0)),
                  pl.BlockSpec((512, 16), lambda i: (0, 0)),
                  pl.BlockSpec((512, 16), lambda i: (0, 0))],
        out_specs=[pl.BlockSpec((bm, 512), lambda i: (i, 0)),
                   pl.BlockSpec((bm, 16), lambda i: (i, 0)),
                   pl.BlockSpec((bm, 16), lambda i: (i, 0))],
        out_shape=[jax.ShapeDtypeStruct((_NP, 512), jnp.float32),
                   jax.ShapeDtypeStruct((_NP, 16), jnp.float32),
                   jax.ShapeDtypeStruct((_NP, 16), jnp.float32)],
    )(xp, W1, asw, adw)


def _mid_body(p0_ref, p1_ref, b1_ref, w2_ref, h2_ref):
    m = (p0_ref[...] + p1_ref[...]) * (1.0 / _HEADS) + b1_ref[...]
    h = jnp.where(m > 0.0, m, jnp.exp(jnp.minimum(m, 0.0)) - 1.0)
    h2_ref[...] = jnp.dot(h, w2_ref[...], preferred_element_type=jnp.float32)


def _mid(part, b1r, w2e, bm=1024):
    nb = _NP // bm
    return pl.pallas_call(
        _mid_body,
        grid=(nb,),
        in_specs=[pl.BlockSpec((bm, _HID), lambda i: (i, 0)),
                  pl.BlockSpec((bm, _HID), lambda i, _nb=nb: (i + _nb, 0)),
                  pl.BlockSpec((1, _HID), lambda i: (0, 0)),
                  pl.BlockSpec((_HID, _C2), lambda i: (0, 0))],
        out_specs=pl.BlockSpec((bm, _C2), lambda i: (i, 0)),
        out_shape=jax.ShapeDtypeStruct((_NP, _C2), jnp.float32),
    )(part, part, b1r, w2e)


def _fin_body(p0_ref, p1_ref, b2_ref, o_ref):
    x = p0_ref[...] + p1_ref[...] + b2_ref[...]
    col = lax.broadcasted_iota(jnp.int32, x.shape, 1)
    x = jnp.where(col < _NCLS, x, -1e30)
    m = jnp.max(x, axis=1, keepdims=True)
    lse = jnp.log(jnp.sum(jnp.exp(x - m), axis=1, keepdims=True))
    o_ref[...] = x - m - lse


def _fin(part2, b2r, bm=1024):
    nb = _NP // bm
    return pl.pallas_call(
        _fin_body,
        grid=(nb,),
        in_specs=[pl.BlockSpec((bm, _C2), lambda i: (i, 0)),
                  pl.BlockSpec((bm, _C2), lambda i, _nb=nb: (i + _nb, 0)),
                  pl.BlockSpec((1, _C2), lambda i: (0, 0))],
        out_specs=pl.BlockSpec((bm, _C2), lambda i: (i, 0)),
        out_shape=jax.ShapeDtypeStruct((_NP, _C2), jnp.float32),
    )(part2, part2, b2r)


def _l1_body(src_h, dst_h, ht_h, ast_h, adt_h,
             part_h, recip_h,
             idxs, idxd, ra, rb, rr, exv, hrows, vout,
             dchunk, denom_sh, acc_sh, sem):
    c = lax.axis_index("c")
    s = lax.axis_index("s")

    def zded(i, _):
        dchunk[i, :] = jnp.zeros((16,), jnp.float32)
        return _
    lax.fori_loop(0, _NPT, zded, None)
    pltpu.sync_copy(dchunk, denom_sh.at[pl.ds(s * _NPT, _NPT)])

    def zv(i, _):
        vout[i // 4, pl.ds((i % 4) * 16, 16)] = jnp.zeros((16,), jnp.float32)
        return _
    lax.fori_loop(0, _K1 * 4, zv, None)
    for kk in range(_NPT // _K1):
        pltpu.sync_copy(vout, acc_sh.at[pl.ds(s * _NPT + kk * _K1, _K1)])
    plsc.subcore_barrier()

    def p1(i, _):
        base = s * _P1T + i * _K1
        pltpu.sync_copy(src_h.at[pl.ds(base, _K1)], idxs)
        pltpu.sync_copy(dst_h.at[pl.ds(base, _K1)], idxd)
        pltpu.async_copy(ast_h.at[idxs], ra, sem).wait()
        pltpu.async_copy(adt_h.at[idxd], rb, sem).wait()

        def cmp(e, _2):
            x = ra[e, :] + rb[e, :]
            exv[e, :] = jnp.exp(_leaky(x))
            return _2
        lax.fori_loop(0, _K1, cmp, None)
        pltpu.sync_copy(exv, denom_sh.at[idxd], add=True)
        return _
    lax.fori_loop(0, _N11, p1, None)
    plsc.subcore_barrier()

    pltpu.sync_copy(denom_sh.at[pl.ds(s * _NPT, _NPT)], dchunk)

    def rec(i, _):
        v = dchunk[i, :]
        dchunk[i, :] = 1.0 / (v + 1e-16)
        return _
    lax.fori_loop(0, _NPT, rec, None)
    pltpu.sync_copy(dchunk, recip_h.at[pl.ds(s * _NPT, _NPT)])
    plsc.subcore_barrier()

    def p2(i, _):
        base = c * (_EEP // 2) + s * _P2T + i * _K1
        pltpu.sync_copy(src_h.at[pl.ds(base, _K1)], idxs)
        pltpu.sync_copy(dst_h.at[pl.ds(base, _K1)], idxd)
        pltpu.async_copy(ast_h.at[idxs], ra, sem).wait()
        pltpu.async_copy(adt_h.at[idxd], rb, sem).wait()
        pltpu.async_copy(recip_h.at[idxd], rr, sem).wait()
        pltpu.async_copy(ht_h.at[idxs], hrows, sem).wait()

        def cmp(e, _2):
            x = ra[e, :] + rb[e, :]
            cf = jnp.exp(_leaky(x)) * rr[e, :]
            accs = [jnp.zeros((16,), jnp.float32) for _j in range(4)]
            for h in range(_HEADS):
                bc = _vbcast(cf, h)
                for j in range(4):
                    accs[j] = accs[j] + bc * hrows[e, pl.ds(h * 64 + j * 16, 16)]
            for j in range(4):
                vout[e, pl.ds(j * 16, 16)] = accs[j]
            return _2
        lax.fori_loop(0, _K1, cmp, None)
        pltpu.sync_copy(vout, acc_sh.at[idxd], add=True)
        return _
    lax.fori_loop(0, _N21, p2, None)
    plsc.subcore_barrier()

    for kk in range(_NPT // _K1):
        pltpu.sync_copy(acc_sh.at[pl.ds(s * _NPT + kk * _K1, _K1)],
                        part_h.at[pl.ds(c * _NP + s * _NPT + kk * _K1, _K1)])


def _sc_layer1(srcd, dstd, ht, ast, adt):
    mesh = plsc.VectorSubcoreMesh(core_axis_name="c", subcore_axis_name="s")
    f = pl.kernel(
        _l1_body,
        out_type=[jax.ShapeDtypeStruct((2 * _NP, _HID), jnp.float32),
                  jax.ShapeDtypeStruct((_NP, 16), jnp.float32)],
        mesh=mesh,
        compiler_params=pltpu.CompilerParams(use_tc_tiling_on_sc=False, needs_layout_passes=False),
        scratch_types=[
            pltpu.VMEM((_K1,), jnp.int32),
            pltpu.VMEM((_K1,), jnp.int32),
            pltpu.VMEM((_K1, 16), jnp.float32),
            pltpu.VMEM((_K1, 16), jnp.float32),
            pltpu.VMEM((_K1, 16), jnp.float32),
            pltpu.VMEM((_K1, 16), jnp.float32),
            pltpu.VMEM((_K1, 512), jnp.float32),
            pltpu.VMEM((_K1, _HID), jnp.float32),
            pltpu.VMEM((_NPT, 16), jnp.float32),
            pltpu.VMEM_SHARED((_NP, 16), jnp.float32),
            pltpu.VMEM_SHARED((_NP, _HID), jnp.float32),
            pltpu.SemaphoreType.DMA,
        ],
    )
    return f(srcd, dstd, ht, ast, adt)


def _l2_body(src_h, dst_h, h2t_h, as2_h, ad2_h,
             part2_h,
             idxs, idxd, as2v, ad2v, rc2, exb, cfb, h2rows, vout2,
             d2chunk, denom2_sh, acc2_sh, sem):
    c = lax.axis_index("c")
    s = lax.axis_index("s")

    pltpu.sync_copy(as2_h, as2v)
    pltpu.sync_copy(ad2_h, ad2v)

    def zd(i, _):
        d2chunk[pl.ds(i * 16, 16)] = jnp.zeros((16,), jnp.float32)
        return _
    lax.fori_loop(0, _NPT // 16, zd, None)
    pltpu.sync_copy(d2chunk, denom2_sh.at[pl.ds(s * _NPT, _NPT)])

    def zv(i, _):
        vout2[i // 3, pl.ds((i % 3) * 16, 16)] = jnp.zeros((16,), jnp.float32)
        return _
    lax.fori_loop(0, _K * 3, zv, None)
    for kk in range(_NPT // _K):
        pltpu.sync_copy(vout2, acc2_sh.at[pl.ds(s * _NPT + kk * _K, _K)])
    plsc.subcore_barrier()

    def p1(i, _):
        base = s * _P1T + i * _K
        pltpu.sync_copy(src_h.at[pl.ds(base, _K)], idxs)
        pltpu.sync_copy(dst_h.at[pl.ds(base, _K)], idxd)

        def cmp(t, _2):
            sv = idxs[pl.ds(t * 16, 16)]
            dv = idxd[pl.ds(t * 16, 16)]
            a = plsc.load_gather(as2v, [sv])
            b = plsc.load_gather(ad2v, [dv])
            exb[pl.ds(t * 16, 16)] = jnp.exp(_leaky(a + b))
            return _2
        lax.fori_loop(0, _K // 16, cmp, None)
        pltpu.sync_copy(exb, denom2_sh.at[idxd], add=True)
        return _
    lax.fori_loop(0, _N1, p1, None)
    plsc.subcore_barrier()

    pltpu.sync_copy(denom2_sh.at[pl.ds(s * _NPT, _NPT)], d2chunk)

    def rec(i, _):
        v = d2chunk[pl.ds(i * 16, 16)]
        d2chunk[pl.ds(i * 16, 16)] = 1.0 / (v + 1e-16)
        return _
    lax.fori_loop(0, _NPT // 16, rec, None)
    pltpu.sync_copy(d2chunk, denom2_sh.at[pl.ds(s * _NPT, _NPT)])
    plsc.subcore_barrier()
    pltpu.sync_copy(denom2_sh, rc2)

    def p2(i, _):
        base = c * (_EEP // 2) + s * _P2T + i * _K
        pltpu.sync_copy(src_h.at[pl.ds(base, _K)], idxs)
        pltpu.sync_copy(dst_h.at[pl.ds(base, _K)], idxd)
        pltpu.async_copy(h2t_h.at[idxs], h2rows, sem).wait()

        def cmp(t, _2):
            sv = idxs[pl.ds(t * 16, 16)]
            dv = idxd[pl.ds(t * 16, 16)]
            a = plsc.load_gather(as2v, [sv])
            b = plsc.load_gather(ad2v, [dv])
            r = plsc.load_gather(rc2, [dv])
            cfb[pl.ds(t * 16, 16)] = jnp.exp(_leaky(a + b)) * r
            return _2
        lax.fori_loop(0, _K // 16, cmp, None)

        def rowm(e, _2):
            bc = plsc.load_gather(cfb, [jnp.full((16,), 0, jnp.int32) + e])
            for j in range(3):
                vout2[e, pl.ds(j * 16, 16)] = bc * h2rows[e, pl.ds(j * 16, 16)]
            return _2
        lax.fori_loop(0, _K, rowm, None)
        pltpu.sync_copy(vout2, acc2_sh.at[idxd], add=True)
        return _
    lax.fori_loop(0, _N2, p2, None)
    plsc.subcore_barrier()

    for kk in range(_NPT // _K):
        pltpu.sync_copy(acc2_sh.at[pl.ds(s * _NPT + kk * _K, _K)],
                        part2_h.at[pl.ds(c * _NP + s * _NPT + kk * _K, _K)])


def _sc_layer2(srcd, dstd, h2t, as2v, ad2v):
    mesh = plsc.VectorSubcoreMesh(core_axis_name="c", subcore_axis_name="s")
    f = pl.kernel(
        _l2_body,
        out_type=jax.ShapeDtypeStruct((2 * _NP, _C2), jnp.float32),
        mesh=mesh,
        compiler_params=pltpu.CompilerParams(use_tc_tiling_on_sc=False, needs_layout_passes=False),
        scratch_types=[
            pltpu.VMEM((_K,), jnp.int32),
            pltpu.VMEM((_K,), jnp.int32),
            pltpu.VMEM((_NP,), jnp.float32),
            pltpu.VMEM((_NP,), jnp.float32),
            pltpu.VMEM((_NP,), jnp.float32),
            pltpu.VMEM((_K,), jnp.float32),
            pltpu.VMEM((_K,), jnp.float32),
            pltpu.VMEM((_K, _C2), jnp.float32),
            pltpu.VMEM((_K, _C2), jnp.float32),
            pltpu.VMEM((_NPT,), jnp.float32),
            pltpu.VMEM_SHARED((_NP,), jnp.float32),
            pltpu.VMEM_SHARED((_NP, _C2), jnp.float32),
            pltpu.SemaphoreType.DMA,
        ],
    )
    return f(srcd, dstd, h2t, as2v, ad2v)


def kernel(x, edge_index, W1, a_src1, a_dst1, b1, W2, a_src2, a_dst2, b2):
    sl = jnp.arange(_N, dtype=jnp.int32)
    npad = _EEP - _EE
    padidx = _N + (jnp.arange(npad, dtype=jnp.int32) % 240)
    srcd = jnp.concatenate([edge_index[0].astype(jnp.int32), sl, padidx])
    dstd = jnp.concatenate([edge_index[1].astype(jnp.int32), sl, padidx])

    rows = jnp.arange(512, dtype=jnp.int32)
    hcol = rows // _HID
    asw = jnp.zeros((512, 16), jnp.float32).at[rows, hcol].set(
        a_src1.reshape(512))
    adw = jnp.zeros((512, 16), jnp.float32).at[rows, hcol].set(
        a_dst1.reshape(512))

    xp = jnp.pad(x, ((0, _NP - _N), (0, 0)))
    ht, ast, adt = _mm1(xp, W1, asw, adw)

    part, _recip = _sc_layer1(srcd, dstd, ht, ast, adt)

    w_as2 = (W2 @ a_src2[0]).reshape(_HID, 1)
    w_ad2 = (W2 @ a_dst2[0]).reshape(_HID, 1)
    w2e = jnp.concatenate(
        [W2, w_as2, w_ad2, jnp.zeros((_HID, _C2 - _NCLS - 2), jnp.float32)],
        axis=1)
    b1r = b1.reshape(1, _HID)
    h2t = _mid(part, b1r, w2e)
    as2v = h2t[:, _NCLS]
    ad2v = h2t[:, _NCLS + 1]

    part2 = _sc_layer2(srcd, dstd, h2t, as2v, ad2v)

    b2r = jnp.pad(b2, (0, _C2 - _NCLS)).reshape(1, _C2)
    o = _fin(part2, b2r)
    return o[:_N, :_NCLS]


# speedup vs baseline: 29.3497x; 1.4546x over previous
import functools

import jax
import jax.numpy as jnp
from jax import lax
import numpy as np
from jax.experimental import pallas as pl
from jax.experimental.pallas import tpu as pltpu
from jax.experimental.pallas import tpu_sc as plsc

_N = 10000
_E = 320000
_D = 128
_HID = 64
_HEADS = 8
_NCLS = 40

_NP = 10240
_NPT = _NP // 16
_K = 128
_K1 = 64
_EE = _E + _N
_EEP = 331776
_P1T = _EEP // 16
_P2T = _EEP // 32
_N1 = _P1T // _K
_N2 = _P2T // _K
_N11 = _P1T // _K1
_N21 = _P2T // _K1
_C2 = 48


def _leaky(x):
    return jnp.where(x >= 0.0, x, 0.2 * x)


_GDN = lax.GatherDimensionNumbers(offset_dims=(), collapsed_slice_dims=(0,),
                                  start_index_map=(0,))


def _vbcast(v, lane):
    idx = jnp.full((16, 1), lane, jnp.int32)
    return lax.gather(v, idx, _GDN, (1,),
                      mode=lax.GatherScatterMode.PROMISE_IN_BOUNDS)


def _mm1_body(x_ref, w_ref, asw_ref, adw_ref, h_ref, as_ref, ad_ref):
    h = jnp.dot(x_ref[...], w_ref[...], preferred_element_type=jnp.float32)
    h_ref[...] = h
    as_ref[...] = jnp.dot(h, asw_ref[...], preferred_element_type=jnp.float32)
    ad_ref[...] = jnp.dot(h, adw_ref[...], preferred_element_type=jnp.float32)


def _mm1(xp, W1, asw, adw, bm=1024):
    return pl.pallas_call(
        _mm1_body,
        grid=(_NP // bm,),
        in_specs=[pl.BlockSpec((bm, _D), lambda i: (i, 0)),
                  pl.BlockSpec((_D, 512), lambda i: (0, 0)),
                  pl.BlockSpec((512, 16), lambda i: (0, 0)),
                  pl.BlockSpec((512, 16), lambda i: (0, 0))],
        out_specs=[pl.BlockSpec((bm, 512), lambda i: (i, 0)),
                   pl.BlockSpec((bm, 16), lambda i: (i, 0)),
                   pl.BlockSpec((bm, 16), lambda i: (i, 0))],
        out_shape=[jax.ShapeDtypeStruct((_NP, 512), jnp.float32),
                   jax.ShapeDtypeStruct((_NP, 16), jnp.float32),
                   jax.ShapeDtypeStruct((_NP, 16), jnp.float32)],
    )(xp, W1, asw, adw)


def _mid_body(p0_ref, p1_ref, b1_ref, w2_ref, h2_ref):
    m = (p0_ref[...] + p1_ref[...]) * (1.0 / _HEADS) + b1_ref[...]
    h = jnp.where(m > 0.0, m, jnp.exp(jnp.minimum(m, 0.0)) - 1.0)
    h2_ref[...] = jnp.dot(h, w2_ref[...], preferred_element_type=jnp.float32)


def _mid(part, b1r, w2e, bm=1024):
    nb = _NP // bm
    return pl.pallas_call(
        _mid_body,
        grid=(nb,),
        in_specs=[pl.BlockSpec((bm, _HID), lambda i: (i, 0)),
                  pl.BlockSpec((bm, _HID), lambda i, _nb=nb: (i + _nb, 0)),
                  pl.BlockSpec((1, _HID), lambda i: (0, 0)),
                  pl.BlockSpec((_HID, _C2), lambda i: (0, 0))],
        out_specs=pl.BlockSpec((bm, _C2), lambda i: (i, 0)),
        out_shape=jax.ShapeDtypeStruct((_NP, _C2), jnp.float32),
    )(part, part, b1r, w2e)


def _fin_body(p0_ref, p1_ref, b2_ref, o_ref):
    x = p0_ref[...] + p1_ref[...] + b2_ref[...]
    col = lax.broadcasted_iota(jnp.int32, x.shape, 1)
    x = jnp.where(col < _NCLS, x, -1e30)
    m = jnp.max(x, axis=1, keepdims=True)
    lse = jnp.log(jnp.sum(jnp.exp(x - m), axis=1, keepdims=True))
    o_ref[...] = x - m - lse


def _fin(part2, b2r, bm=1024):
    nb = _NP // bm
    return pl.pallas_call(
        _fin_body,
        grid=(nb,),
        in_specs=[pl.BlockSpec((bm, _C2), lambda i: (i, 0)),
                  pl.BlockSpec((bm, _C2), lambda i, _nb=nb: (i + _nb, 0)),
                  pl.BlockSpec((1, _C2), lambda i: (0, 0))],
        out_specs=pl.BlockSpec((bm, _C2), lambda i: (i, 0)),
        out_shape=jax.ShapeDtypeStruct((_NP, _C2), jnp.float32),
    )(part2, part2, b2r)


def _l1_body(src_h, dst_h, ht_h, ast_h, adt_h,
             part_h, recip_h,
             idxs0, idxd0, idxs1, idxd1,
             ra0, rb0, rr0, ra1, rb1, rr1, h0, h1b, vout, strip,
             denom_sh, acc_sh, sem0, sem1):
    c = lax.axis_index("c")
    s = lax.axis_index("s")
    idxsl, idxdl = (idxs0, idxs1), (idxd0, idxd1)
    ral, rbl, rrl, hl = (ra0, ra1), (rb0, rb1), (rr0, rr1), (h0, h1b)
    seml = (sem0, sem1)

    def zs(i, _):
        strip[i, :] = jnp.zeros((16,), jnp.float32)
        return _
    lax.fori_loop(0, 128, zs, None)
    for kk in range(_NPT // 128):
        pltpu.sync_copy(strip, denom_sh.at[pl.ds(s * _NPT + kk * 128, 128)])

    def zv(i, _):
        vout[i // 4, pl.ds((i % 4) * 16, 16)] = jnp.zeros((16,), jnp.float32)
        return _
    lax.fori_loop(0, _K1 * 4, zv, None)
    for kk in range(_NPT // _K1):
        pltpu.sync_copy(vout, acc_sh.at[pl.ds(s * _NPT + kk * _K1, _K1)])
    plsc.subcore_barrier()

    def p1_start(bb, i):
        base = s * _P1T + i * _K1
        pltpu.sync_copy(src_h.at[pl.ds(base, _K1)], idxsl[bb])
        pltpu.sync_copy(dst_h.at[pl.ds(base, _K1)], idxdl[bb])
        pltpu.async_copy(ast_h.at[idxsl[bb]], ral[bb], seml[bb])
        pltpu.async_copy(adt_h.at[idxdl[bb]], rbl[bb], seml[bb])

    def p1_fin(bb):
        pltpu.make_async_copy(ast_h.at[idxsl[bb]], ral[bb], seml[bb]).wait()
        pltpu.make_async_copy(adt_h.at[idxdl[bb]], rbl[bb], seml[bb]).wait()

        def cmp(e, _2):
            x = ral[bb][e, :] + rbl[bb][e, :]
            ral[bb][e, :] = jnp.exp(_leaky(x))
            return _2
        lax.fori_loop(0, _K1, cmp, None)
        pltpu.sync_copy(ral[bb], denom_sh.at[idxdl[bb]], add=True)

    p1_start(0, 0)

    def p1o(j, _):
        for bb in range(2):
            i = 2 * j + bb

            @pl.when(i + 1 < _N11)
            def _start():
                p1_start(1 - bb, i + 1)
            p1_fin(bb)
        return _
    lax.fori_loop(0, _N11 // 2, p1o, None)
    plsc.subcore_barrier()

    for kk in range(_NPT // 128):
        off = s * _NPT + kk * 128
        pltpu.sync_copy(denom_sh.at[pl.ds(off, 128)], strip)

        def rec(i, _):
            v = strip[i, :]
            strip[i, :] = 1.0 / (v + 1e-16)
            return _
        lax.fori_loop(0, 128, rec, None)
        pltpu.sync_copy(strip, recip_h.at[pl.ds(off, 128)])
    plsc.subcore_barrier()

    def p2_start(bb, i):
        base = c * (_EEP // 2) + s * _P2T + i * _K1
        pltpu.sync_copy(src_h.at[pl.ds(base, _K1)], idxsl[bb])
        pltpu.sync_copy(dst_h.at[pl.ds(base, _K1)], idxdl[bb])
        pltpu.async_copy(ast_h.at[idxsl[bb]], ral[bb], seml[bb])
        pltpu.async_copy(adt_h.at[idxdl[bb]], rbl[bb], seml[bb])
        pltpu.async_copy(recip_h.at[idxdl[bb]], rrl[bb], seml[bb])
        pltpu.async_copy(ht_h.at[idxsl[bb]], hl[bb], seml[bb])

    def p2_fin(bb):
        pltpu.make_async_copy(ast_h.at[idxsl[bb]], ral[bb], seml[bb]).wait()
        pltpu.make_async_copy(adt_h.at[idxdl[bb]], rbl[bb], seml[bb]).wait()
        pltpu.make_async_copy(recip_h.at[idxdl[bb]], rrl[bb], seml[bb]).wait()
        pltpu.make_async_copy(ht_h.at[idxsl[bb]], hl[bb], seml[bb]).wait()

        def cmp(e, _2):
            x = ral[bb][e, :] + rbl[bb][e, :]
            cf = jnp.exp(_leaky(x)) * rrl[bb][e, :]
            accs = [jnp.zeros((16,), jnp.float32) for _j in range(4)]
            for h in range(_HEADS):
                bc = _vbcast(cf, h)
                for j in range(4):
                    accs[j] = accs[j] + bc * hl[bb][e, pl.ds(h * 64 + j * 16, 16)]
            for j in range(4):
                vout[e, pl.ds(j * 16, 16)] = accs[j]
            return _2
        lax.fori_loop(0, _K1, cmp, None)
        pltpu.sync_copy(vout, acc_sh.at[idxdl[bb]], add=True)

    p2_start(0, 0)

    def p2o(j, _):
        for bb in range(2):
            i = 2 * j + bb

            @pl.when(i + 1 < _N21)
            def _start():
                p2_start(1 - bb, i + 1)
            p2_fin(bb)
        return _
    lax.fori_loop(0, _N21 // 2, p2o, None)
    plsc.subcore_barrier()

    for kk in range(_NPT // _K1):
        pltpu.sync_copy(acc_sh.at[pl.ds(s * _NPT + kk * _K1, _K1)],
                        part_h.at[pl.ds(c * _NP + s * _NPT + kk * _K1, _K1)])


def _sc_layer1(srcd, dstd, ht, ast, adt):
    mesh = plsc.VectorSubcoreMesh(core_axis_name="c", subcore_axis_name="s")
    f = pl.kernel(
        _l1_body,
        out_type=[jax.ShapeDtypeStruct((2 * _NP, _HID), jnp.float32),
                  jax.ShapeDtypeStruct((_NP, 16), jnp.float32)],
        mesh=mesh,
        compiler_params=pltpu.CompilerParams(use_tc_tiling_on_sc=False, needs_layout_passes=False),
        scratch_types=[
            pltpu.VMEM((_K1,), jnp.int32),
            pltpu.VMEM((_K1,), jnp.int32),
            pltpu.VMEM((_K1,), jnp.int32),
            pltpu.VMEM((_K1,), jnp.int32),
            pltpu.VMEM((_K1, 16), jnp.float32),
            pltpu.VMEM((_K1, 16), jnp.float32),
            pltpu.VMEM((_K1, 16), jnp.float32),
            pltpu.VMEM((_K1, 16), jnp.float32),
            pltpu.VMEM((_K1, 16), jnp.float32),
            pltpu.VMEM((_K1, 16), jnp.float32),
            pltpu.VMEM((_K1, 512), jnp.float32),
            pltpu.VMEM((_K1, 512), jnp.float32),
            pltpu.VMEM((_K1, _HID), jnp.float32),
            pltpu.VMEM((128, 16), jnp.float32),
            pltpu.VMEM_SHARED((_NP, 16), jnp.float32),
            pltpu.VMEM_SHARED((_NP, _HID), jnp.float32),
            pltpu.SemaphoreType.DMA,
            pltpu.SemaphoreType.DMA,
        ],
    )
    return f(srcd, dstd, ht, ast, adt)


def _l2_body(src_h, dst_h, h2t_h, as2_h, ad2_h,
             part2_h,
             idxs, idxd, as2v, ad2v, rc2, exb, cfb, h2rows, vout2,
             d2chunk, denom2_sh, acc2_sh, sem):
    c = lax.axis_index("c")
    s = lax.axis_index("s")

    pltpu.sync_copy(as2_h, as2v)
    pltpu.sync_copy(ad2_h, ad2v)

    def zd(i, _):
        d2chunk[pl.ds(i * 16, 16)] = jnp.zeros((16,), jnp.float32)
        return _
    lax.fori_loop(0, _NPT // 16, zd, None)
    pltpu.sync_copy(d2chunk, denom2_sh.at[pl.ds(s * _NPT, _NPT)])

    def zv(i, _):
        vout2[i // 3, pl.ds((i % 3) * 16, 16)] = jnp.zeros((16,), jnp.float32)
        return _
    lax.fori_loop(0, _K * 3, zv, None)
    for kk in range(_NPT // _K):
        pltpu.sync_copy(vout2, acc2_sh.at[pl.ds(s * _NPT + kk * _K, _K)])
    plsc.subcore_barrier()

    def p1(i, _):
        base = s * _P1T + i * _K
        pltpu.sync_copy(src_h.at[pl.ds(base, _K)], idxs)
        pltpu.sync_copy(dst_h.at[pl.ds(base, _K)], idxd)

        def cmp(t, _2):
            sv = idxs[pl.ds(t * 16, 16)]
            dv = idxd[pl.ds(t * 16, 16)]
            a = plsc.load_gather(as2v, [sv])
            b = plsc.load_gather(ad2v, [dv])
            exb[pl.ds(t * 16, 16)] = jnp.exp(_leaky(a + b))
            return _2
        lax.fori_loop(0, _K // 16, cmp, None)
        pltpu.sync_copy(exb, denom2_sh.at[idxd], add=True)
        return _
    lax.fori_loop(0, _N1, p1, None)
    plsc.subcore_barrier()

    pltpu.sync_copy(denom2_sh.at[pl.ds(s * _NPT, _NPT)], d2chunk)

    def rec(i, _):
        v = d2chunk[pl.ds(i * 16, 16)]
        d2chunk[pl.ds(i * 16, 16)] = 1.0 / (v + 1e-16)
        return _
    lax.fori_loop(0, _NPT // 16, rec, None)
    pltpu.sync_copy(d2chunk, denom2_sh.at[pl.ds(s * _NPT, _NPT)])
    plsc.subcore_barrier()
    pltpu.sync_copy(denom2_sh, rc2)

    def p2(i, _):
        base = c * (_EEP // 2) + s * _P2T + i * _K
        pltpu.sync_copy(src_h.at[pl.ds(base, _K)], idxs)
        pltpu.sync_copy(dst_h.at[pl.ds(base, _K)], idxd)
        pltpu.async_copy(h2t_h.at[idxs], h2rows, sem).wait()

        def cmp(t, _2):
            sv = idxs[pl.ds(t * 16, 16)]
            dv = idxd[pl.ds(t * 16, 16)]
            a = plsc.load_gather(as2v, [sv])
            b = plsc.load_gather(ad2v, [dv])
            r = plsc.load_gather(rc2, [dv])
            cfb[pl.ds(t * 16, 16)] = jnp.exp(_leaky(a + b)) * r
            return _2
        lax.fori_loop(0, _K // 16, cmp, None)

        def rowm(e, _2):
            bc = plsc.load_gather(cfb, [jnp.full((16,), 0, jnp.int32) + e])
            for j in range(3):
                vout2[e, pl.ds(j * 16, 16)] = bc * h2rows[e, pl.ds(j * 16, 16)]
            return _2
        lax.fori_loop(0, _K, rowm, None)
        pltpu.sync_copy(vout2, acc2_sh.at[idxd], add=True)
        return _
    lax.fori_loop(0, _N2, p2, None)
    plsc.subcore_barrier()

    for kk in range(_NPT // _K):
        pltpu.sync_copy(acc2_sh.at[pl.ds(s * _NPT + kk * _K, _K)],
                        part2_h.at[pl.ds(c * _NP + s * _NPT + kk * _K, _K)])


def _sc_layer2(srcd, dstd, h2t, as2v, ad2v):
    mesh = plsc.VectorSubcoreMesh(core_axis_name="c", subcore_axis_name="s")
    f = pl.kernel(
        _l2_body,
        out_type=jax.ShapeDtypeStruct((2 * _NP, _C2), jnp.float32),
        mesh=mesh,
        compiler_params=pltpu.CompilerParams(use_tc_tiling_on_sc=False, needs_layout_passes=False),
        scratch_types=[
            pltpu.VMEM((_K,), jnp.int32),
            pltpu.VMEM((_K,), jnp.int32),
            pltpu.VMEM((_NP,), jnp.float32),
            pltpu.VMEM((_NP,), jnp.float32),
            pltpu.VMEM((_NP,), jnp.float32),
            pltpu.VMEM((_K,), jnp.float32),
            pltpu.VMEM((_K,), jnp.float32),
            pltpu.VMEM((_K, _C2), jnp.float32),
            pltpu.VMEM((_K, _C2), jnp.float32),
            pltpu.VMEM((_NPT,), jnp.float32),
            pltpu.VMEM_SHARED((_NP,), jnp.float32),
            pltpu.VMEM_SHARED((_NP, _C2), jnp.float32),
            pltpu.SemaphoreType.DMA,
        ],
    )
    return f(srcd, dstd, h2t, as2v, ad2v)


def kernel(x, edge_index, W1, a_src1, a_dst1, b1, W2, a_src2, a_dst2, b2):
    sl = jnp.arange(_N, dtype=jnp.int32)
    npad = _EEP - _EE
    padidx = _N + (jnp.arange(npad, dtype=jnp.int32) % 240)
    srcd = jnp.concatenate([edge_index[0].astype(jnp.int32), sl, padidx])
    dstd = jnp.concatenate([edge_index[1].astype(jnp.int32), sl, padidx])

    rows = jnp.arange(512, dtype=jnp.int32)
    hcol = rows // _HID
    asw = jnp.zeros((512, 16), jnp.float32).at[rows, hcol].set(
        a_src1.reshape(512))
    adw = jnp.zeros((512, 16), jnp.float32).at[rows, hcol].set(
        a_dst1.reshape(512))

    xp = jnp.pad(x, ((0, _NP - _N), (0, 0)))
    ht, ast, adt = _mm1(xp, W1, asw, adw)

    part, _recip = _sc_layer1(srcd, dstd, ht, ast, adt)

    w_as2 = (W2 @ a_src2[0]).reshape(_HID, 1)
    w_ad2 = (W2 @ a_dst2[0]).reshape(_HID, 1)
    w2e = jnp.concatenate(
        [W2, w_as2, w_ad2, jnp.zeros((_HID, _C2 - _NCLS - 2), jnp.float32)],
        axis=1)
    b1r = b1.reshape(1, _HID)
    h2t = _mid(part, b1r, w2e)
    as2v = h2t[:, _NCLS]
    ad2v = h2t[:, _NCLS + 1]

    part2 = _sc_layer2(srcd, dstd, h2t, as2v, ad2v)

    b2r = jnp.pad(b2, (0, _C2 - _NCLS)).reshape(1, _C2)
    o = _fin(part2, b2r)
    return o[:_N, :_NCLS]


# speedup vs baseline: 33.6488x; 1.1465x over previous
import functools

import jax
import jax.numpy as jnp
from jax import lax
import numpy as np
from jax.experimental import pallas as pl
from jax.experimental.pallas import tpu as pltpu
from jax.experimental.pallas import tpu_sc as plsc

_N = 10000
_E = 320000
_D = 128
_HID = 64
_HEADS = 8
_NCLS = 40

_NP = 10240
_NPT = _NP // 16
_K = 128
_K1 = 64
_EE = _E + _N
_EEP = 331776
_P1T = _EEP // 16
_P2T = _EEP // 32
_N1 = _P1T // _K
_N2 = _P2T // _K
_N11 = _P1T // _K1
_N21 = _P2T // _K1
_C2 = 48


def _leaky(x):
    return jnp.where(x >= 0.0, x, 0.2 * x)


_GDN = lax.GatherDimensionNumbers(offset_dims=(), collapsed_slice_dims=(0,),
                                  start_index_map=(0,))


def _vbcast(v, lane):
    idx = jnp.full((16, 1), lane, jnp.int32)
    return lax.gather(v, idx, _GDN, (1,),
                      mode=lax.GatherScatterMode.PROMISE_IN_BOUNDS)


def _mm1_body(x_ref, w_ref, asw_ref, adw_ref, h_ref, as_ref, ad_ref):
    h = jnp.dot(x_ref[...], w_ref[...], preferred_element_type=jnp.float32)
    h_ref[...] = h
    as_ref[...] = jnp.dot(h, asw_ref[...], preferred_element_type=jnp.float32)
    ad_ref[...] = jnp.dot(h, adw_ref[...], preferred_element_type=jnp.float32)


def _mm1(xp, W1, asw, adw, bm=1024):
    return pl.pallas_call(
        _mm1_body,
        grid=(_NP // bm,),
        in_specs=[pl.BlockSpec((bm, _D), lambda i: (i, 0)),
                  pl.BlockSpec((_D, 512), lambda i: (0, 0)),
                  pl.BlockSpec((512, 16), lambda i: (0, 0)),
                  pl.BlockSpec((512, 16), lambda i: (0, 0))],
        out_specs=[pl.BlockSpec((bm, 512), lambda i: (i, 0)),
                   pl.BlockSpec((bm, 16), lambda i: (i, 0)),
                   pl.BlockSpec((bm, 16), lambda i: (i, 0))],
        out_shape=[jax.ShapeDtypeStruct((_NP, 512), jnp.float32),
                   jax.ShapeDtypeStruct((_NP, 16), jnp.float32),
                   jax.ShapeDtypeStruct((_NP, 16), jnp.float32)],
    )(xp, W1, asw, adw)


def _mid_body(p0_ref, p1_ref, b1_ref, w2_ref, h2_ref):
    m = (p0_ref[...] + p1_ref[...]) * (1.0 / _HEADS) + b1_ref[...]
    h = jnp.where(m > 0.0, m, jnp.exp(jnp.minimum(m, 0.0)) - 1.0)
    h2_ref[...] = jnp.dot(h, w2_ref[...], preferred_element_type=jnp.float32)


def _mid(part, b1r, w2e, bm=1024):
    nb = _NP // bm
    return pl.pallas_call(
        _mid_body,
        grid=(nb,),
        in_specs=[pl.BlockSpec((bm, _HID), lambda i: (i, 0)),
                  pl.BlockSpec((bm, _HID), lambda i, _nb=nb: (i + _nb, 0)),
                  pl.BlockSpec((1, _HID), lambda i: (0, 0)),
                  pl.BlockSpec((_HID, _C2), lambda i: (0, 0))],
        out_specs=pl.BlockSpec((bm, _C2), lambda i: (i, 0)),
        out_shape=jax.ShapeDtypeStruct((_NP, _C2), jnp.float32),
    )(part, part, b1r, w2e)


def _fin_body(p0_ref, p1_ref, b2_ref, o_ref):
    x = p0_ref[...] + p1_ref[...] + b2_ref[...]
    col = lax.broadcasted_iota(jnp.int32, x.shape, 1)
    x = jnp.where(col < _NCLS, x, -1e30)
    m = jnp.max(x, axis=1, keepdims=True)
    lse = jnp.log(jnp.sum(jnp.exp(x - m), axis=1, keepdims=True))
    o_ref[...] = x - m - lse


def _fin(part2, b2r, bm=1024):
    nb = _NP // bm
    return pl.pallas_call(
        _fin_body,
        grid=(nb,),
        in_specs=[pl.BlockSpec((bm, _C2), lambda i: (i, 0)),
                  pl.BlockSpec((bm, _C2), lambda i, _nb=nb: (i + _nb, 0)),
                  pl.BlockSpec((1, _C2), lambda i: (0, 0))],
        out_specs=pl.BlockSpec((bm, _C2), lambda i: (i, 0)),
        out_shape=jax.ShapeDtypeStruct((_NP, _C2), jnp.float32),
    )(part2, part2, b2r)


def _l1_body(src_h, dst_h, ht_h, ast_h, adt_h,
             part_h, recip_h,
             idxs0, idxd0, idxs1, idxd1,
             ra0, rb0, rr0, ra1, rb1, rr1, h0, h1b, vout, strip,
             denom_sh, acc_sh, sem0, sem1):
    c = lax.axis_index("c")
    s = lax.axis_index("s")
    idxsl, idxdl = (idxs0, idxs1), (idxd0, idxd1)
    ral, rbl, rrl, hl = (ra0, ra1), (rb0, rb1), (rr0, rr1), (h0, h1b)
    seml = (sem0, sem1)

    def zs(i, _):
        strip[i, :] = jnp.zeros((16,), jnp.float32)
        return _
    lax.fori_loop(0, 128, zs, None)
    for kk in range(_NPT // 128):
        pltpu.sync_copy(strip, denom_sh.at[pl.ds(s * _NPT + kk * 128, 128)])

    def zv(i, _):
        vout[i // 4, pl.ds((i % 4) * 16, 16)] = jnp.zeros((16,), jnp.float32)
        return _
    lax.fori_loop(0, _K1 * 4, zv, None)
    for kk in range(_NPT // _K1):
        pltpu.sync_copy(vout, acc_sh.at[pl.ds(s * _NPT + kk * _K1, _K1)])
    plsc.subcore_barrier()

    def p1_start(bb, i):
        base = s * _P1T + i * _K1
        pltpu.sync_copy(src_h.at[pl.ds(base, _K1)], idxsl[bb])
        pltpu.sync_copy(dst_h.at[pl.ds(base, _K1)], idxdl[bb])
        pltpu.async_copy(ast_h.at[idxsl[bb]], ral[bb], seml[bb])
        pltpu.async_copy(adt_h.at[idxdl[bb]], rbl[bb], seml[bb])

    def p1_fin(bb):
        pltpu.make_async_copy(ast_h.at[idxsl[bb]], ral[bb], seml[bb]).wait()
        pltpu.make_async_copy(adt_h.at[idxdl[bb]], rbl[bb], seml[bb]).wait()

        def cmp(e, _2):
            x = ral[bb][e, :] + rbl[bb][e, :]
            ral[bb][e, :] = jnp.exp(_leaky(x))
            return _2
        lax.fori_loop(0, _K1, cmp, None)
        pltpu.sync_copy(ral[bb], denom_sh.at[idxdl[bb]], add=True)

    p1_start(0, 0)

    def p1o(j, _):
        for bb in range(2):
            i = 2 * j + bb

            @pl.when(i + 1 < _N11)
            def _start():
                p1_start(1 - bb, i + 1)
            p1_fin(bb)
        return _
    lax.fori_loop(0, _N11 // 2, p1o, None)
    plsc.subcore_barrier()

    for kk in range(_NPT // 128):
        off = s * _NPT + kk * 128
        pltpu.sync_copy(denom_sh.at[pl.ds(off, 128)], strip)

        def rec(i, _):
            v = strip[i, :]
            strip[i, :] = 1.0 / (v + 1e-16)
            return _
        lax.fori_loop(0, 128, rec, None)
        pltpu.sync_copy(strip, recip_h.at[pl.ds(off, 128)])
    plsc.subcore_barrier()

    def p2_start(bb, i):
        base = c * (_EEP // 2) + s * _P2T + i * _K1
        pltpu.sync_copy(src_h.at[pl.ds(base, _K1)], idxsl[bb])
        pltpu.sync_copy(dst_h.at[pl.ds(base, _K1)], idxdl[bb])
        pltpu.async_copy(ast_h.at[idxsl[bb]], ral[bb], seml[bb])
        pltpu.async_copy(adt_h.at[idxdl[bb]], rbl[bb], seml[bb])
        pltpu.async_copy(recip_h.at[idxdl[bb]], rrl[bb], seml[bb])
        pltpu.async_copy(ht_h.at[idxsl[bb]], hl[bb], seml[bb])

    def p2_fin(bb):
        pltpu.make_async_copy(ast_h.at[idxsl[bb]], ral[bb], seml[bb]).wait()
        pltpu.make_async_copy(adt_h.at[idxdl[bb]], rbl[bb], seml[bb]).wait()
        pltpu.make_async_copy(recip_h.at[idxdl[bb]], rrl[bb], seml[bb]).wait()
        pltpu.make_async_copy(ht_h.at[idxsl[bb]], hl[bb], seml[bb]).wait()

        def cmp(e, _2):
            x = ral[bb][e, :] + rbl[bb][e, :]
            cf = jnp.exp(_leaky(x)) * rrl[bb][e, :]
            accs = [jnp.zeros((16,), jnp.float32) for _j in range(4)]
            for h in range(_HEADS):
                bc = _vbcast(cf, h)
                for j in range(4):
                    accs[j] = accs[j] + bc * hl[bb][e, pl.ds(h * 64 + j * 16, 16)]
            for j in range(4):
                vout[e, pl.ds(j * 16, 16)] = accs[j]
            return _2
        lax.fori_loop(0, _K1, cmp, None)
        pltpu.sync_copy(vout, acc_sh.at[idxdl[bb]], add=True)

    p2_start(0, 0)

    def p2o(j, _):
        for bb in range(2):
            i = 2 * j + bb

            @pl.when(i + 1 < _N21)
            def _start():
                p2_start(1 - bb, i + 1)
            p2_fin(bb)
        return _
    lax.fori_loop(0, _N21 // 2, p2o, None)
    plsc.subcore_barrier()

    for kk in range(_NPT // _K1):
        pltpu.sync_copy(acc_sh.at[pl.ds(s * _NPT + kk * _K1, _K1)],
                        part_h.at[pl.ds(c * _NP + s * _NPT + kk * _K1, _K1)])


def _sc_layer1(srcd, dstd, ht, ast, adt):
    mesh = plsc.VectorSubcoreMesh(core_axis_name="c", subcore_axis_name="s")
    f = pl.kernel(
        _l1_body,
        out_type=[jax.ShapeDtypeStruct((2 * _NP, _HID), jnp.float32),
                  jax.ShapeDtypeStruct((_NP, 16), jnp.float32)],
        mesh=mesh,
        compiler_params=pltpu.CompilerParams(use_tc_tiling_on_sc=False, needs_layout_passes=False),
        scratch_types=[
            pltpu.VMEM((_K1,), jnp.int32),
            pltpu.VMEM((_K1,), jnp.int32),
            pltpu.VMEM((_K1,), jnp.int32),
            pltpu.VMEM((_K1,), jnp.int32),
            pltpu.VMEM((_K1, 16), jnp.float32),
            pltpu.VMEM((_K1, 16), jnp.float32),
            pltpu.VMEM((_K1, 16), jnp.float32),
            pltpu.VMEM((_K1, 16), jnp.float32),
            pltpu.VMEM((_K1, 16), jnp.float32),
            pltpu.VMEM((_K1, 16), jnp.float32),
            pltpu.VMEM((_K1, 512), jnp.float32),
            pltpu.VMEM((_K1, 512), jnp.float32),
            pltpu.VMEM((_K1, _HID), jnp.float32),
            pltpu.VMEM((128, 16), jnp.float32),
            pltpu.VMEM_SHARED((_NP, 16), jnp.float32),
            pltpu.VMEM_SHARED((_NP, _HID), jnp.float32),
            pltpu.SemaphoreType.DMA,
            pltpu.SemaphoreType.DMA,
        ],
    )
    return f(srcd, dstd, ht, ast, adt)


_K2 = 192
_N12 = _P1T // _K2
_N22 = _P2T // _K2


def _l2_body(src_h, dst_h, h2t_h, as2_h, ad2_h,
             part2_h,
             idxs0, idxd0, idxs1, idxd1, as2v, ad2v, rc2, exb, cfb,
             hr0, hr1, vout2, d2chunk,
             h2_sh, denom2_sh, acc2_sh, sem0, sem1):
    c = lax.axis_index("c")
    s = lax.axis_index("s")
    idxsl, idxdl = (idxs0, idxs1), (idxd0, idxd1)
    hrl, seml = (hr0, hr1), (sem0, sem1)

    pltpu.sync_copy(as2_h, as2v)
    pltpu.sync_copy(ad2_h, ad2v)
    pltpu.sync_copy(h2t_h.at[pl.ds(s * _NPT, _NPT)],
                    h2_sh.at[pl.ds(s * _NPT, _NPT)])

    def zd(i, _):
        d2chunk[pl.ds(i * 16, 16)] = jnp.zeros((16,), jnp.float32)
        return _
    lax.fori_loop(0, _NPT // 16, zd, None)
    pltpu.sync_copy(d2chunk, denom2_sh.at[pl.ds(s * _NPT, _NPT)])

    def zv(i, _):
        vout2[i // 3, pl.ds((i % 3) * 16, 16)] = jnp.zeros((16,), jnp.float32)
        return _
    lax.fori_loop(0, _K2 * 3, zv, None)
    for kk in range(0, _NPT, _K2):
        nrow = min(_K2, _NPT - kk)
        pltpu.sync_copy(vout2.at[pl.ds(0, nrow)],
                        acc2_sh.at[pl.ds(s * _NPT + kk, nrow)])
    plsc.subcore_barrier()

    def p1_start(bb, i):
        base = s * _P1T + i * _K2
        pltpu.async_copy(src_h.at[pl.ds(base, _K2)], idxsl[bb], seml[bb])
        pltpu.async_copy(dst_h.at[pl.ds(base, _K2)], idxdl[bb], seml[bb])

    def p1_fin(bb):
        pltpu.make_async_copy(src_h.at[pl.ds(0, _K2)], idxsl[bb], seml[bb]).wait()
        pltpu.make_async_copy(dst_h.at[pl.ds(0, _K2)], idxdl[bb], seml[bb]).wait()

        def cmp(t, _2):
            sv = idxsl[bb][pl.ds(t * 16, 16)]
            dv = idxdl[bb][pl.ds(t * 16, 16)]
            aa = plsc.load_gather(as2v, [sv])
            ab = plsc.load_gather(ad2v, [dv])
            exb[pl.ds(t * 16, 16)] = jnp.exp(_leaky(aa + ab))
            return _2
        lax.fori_loop(0, _K2 // 16, cmp, None)
        pltpu.sync_copy(exb, denom2_sh.at[idxdl[bb]], add=True)

    p1_start(0, 0)

    def p1o(j, _):
        for bb in range(2):
            i = 2 * j + bb

            @pl.when(i + 1 < _N12)
            def _st():
                p1_start(1 - bb, i + 1)
            p1_fin(bb)
        return _
    lax.fori_loop(0, _N12 // 2, p1o, None)
    plsc.subcore_barrier()

    pltpu.sync_copy(denom2_sh.at[pl.ds(s * _NPT, _NPT)], d2chunk)

    def rec(i, _):
        v = d2chunk[pl.ds(i * 16, 16)]
        d2chunk[pl.ds(i * 16, 16)] = 1.0 / (v + 1e-16)
        return _
    lax.fori_loop(0, _NPT // 16, rec, None)
    pltpu.sync_copy(d2chunk, denom2_sh.at[pl.ds(s * _NPT, _NPT)])
    plsc.subcore_barrier()
    pltpu.sync_copy(denom2_sh, rc2)

    def p2_start(bb, i):
        base = c * (_EEP // 2) + s * _P2T + i * _K2
        pltpu.sync_copy(src_h.at[pl.ds(base, _K2)], idxsl[bb])
        pltpu.sync_copy(dst_h.at[pl.ds(base, _K2)], idxdl[bb])
        pltpu.async_copy(h2_sh.at[idxsl[bb]], hrl[bb], seml[bb])

    def p2_fin(bb):
        pltpu.make_async_copy(h2_sh.at[idxsl[bb]], hrl[bb], seml[bb]).wait()

        def cmp(t, _2):
            sv = idxsl[bb][pl.ds(t * 16, 16)]
            dv = idxdl[bb][pl.ds(t * 16, 16)]
            aa = plsc.load_gather(as2v, [sv])
            ab = plsc.load_gather(ad2v, [dv])
            r = plsc.load_gather(rc2, [dv])
            cfb[pl.ds(t * 16, 16)] = jnp.exp(_leaky(aa + ab)) * r
            return _2
        lax.fori_loop(0, _K2 // 16, cmp, None)

        def rowm(e, _2):
            bc = _vbcast(cfb[pl.ds((e // 16) * 16, 16)], e % 16)
            for j in range(3):
                vout2[e, pl.ds(j * 16, 16)] = bc * hrl[bb][e, pl.ds(j * 16, 16)]
            return _2
        lax.fori_loop(0, _K2, rowm, None)
        pltpu.sync_copy(vout2, acc2_sh.at[idxdl[bb]], add=True)

    p2_start(0, 0)

    def p2o(j, _):
        for bb in range(2):
            i = 2 * j + bb

            @pl.when(i + 1 < _N22)
            def _st():
                p2_start(1 - bb, i + 1)
            p2_fin(bb)
        return _
    lax.fori_loop(0, _N22 // 2, p2o, None)
    plsc.subcore_barrier()

    for kk in range(0, _NPT, _K2):
        nrow = min(_K2, _NPT - kk)
        pltpu.sync_copy(acc2_sh.at[pl.ds(s * _NPT + kk, nrow)],
                        part2_h.at[pl.ds(c * _NP + s * _NPT + kk, nrow)])


def _sc_layer2(srcd, dstd, h2t, as2v, ad2v):
    mesh = plsc.VectorSubcoreMesh(core_axis_name="c", subcore_axis_name="s")
    f = pl.kernel(
        _l2_body,
        out_type=jax.ShapeDtypeStruct((2 * _NP, _C2), jnp.float32),
        mesh=mesh,
        compiler_params=pltpu.CompilerParams(use_tc_tiling_on_sc=False, needs_layout_passes=False),
        scratch_types=[
            pltpu.VMEM((_K2,), jnp.int32),
            pltpu.VMEM((_K2,), jnp.int32),
            pltpu.VMEM((_K2,), jnp.int32),
            pltpu.VMEM((_K2,), jnp.int32),
            pltpu.VMEM((_NP,), jnp.float32),
            pltpu.VMEM((_NP,), jnp.float32),
            pltpu.VMEM((_NP,), jnp.float32),
            pltpu.VMEM((_K2,), jnp.float32),
            pltpu.VMEM((_K2,), jnp.float32),
            pltpu.VMEM((_K2, _C2), jnp.float32),
            pltpu.VMEM((_K2, _C2), jnp.float32),
            pltpu.VMEM((_K2, _C2), jnp.float32),
            pltpu.VMEM((_NPT,), jnp.float32),
            pltpu.VMEM_SHARED((_NP, _C2), jnp.float32),
            pltpu.VMEM_SHARED((_NP,), jnp.float32),
            pltpu.VMEM_SHARED((_NP, _C2), jnp.float32),
            pltpu.SemaphoreType.DMA,
            pltpu.SemaphoreType.DMA,
        ],
    )
    return f(srcd, dstd, h2t, as2v, ad2v)


def kernel(x, edge_index, W1, a_src1, a_dst1, b1, W2, a_src2, a_dst2, b2):
    sl = jnp.arange(_N, dtype=jnp.int32)
    npad = _EEP - _EE
    padidx = _N + (jnp.arange(npad, dtype=jnp.int32) % 240)
    srcd = jnp.concatenate([edge_index[0].astype(jnp.int32), sl, padidx])
    dstd = jnp.concatenate([edge_index[1].astype(jnp.int32), sl, padidx])

    rows = jnp.arange(512, dtype=jnp.int32)
    hcol = rows // _HID
    asw = jnp.zeros((512, 16), jnp.float32).at[rows, hcol].set(
        a_src1.reshape(512))
    adw = jnp.zeros((512, 16), jnp.float32).at[rows, hcol].set(
        a_dst1.reshape(512))

    xp = jnp.pad(x, ((0, _NP - _N), (0, 0)))
    ht, ast, adt = _mm1(xp, W1, asw, adw)

    part, _recip = _sc_layer1(srcd, dstd, ht, ast, adt)

    w_as2 = (W2 @ a_src2[0]).reshape(_HID, 1)
    w_ad2 = (W2 @ a_dst2[0]).reshape(_HID, 1)
    w2e = jnp.concatenate(
        [W2, w_as2, w_ad2, jnp.zeros((_HID, _C2 - _NCLS - 2), jnp.float32)],
        axis=1)
    b1r = b1.reshape(1, _HID)
    h2t = _mid(part, b1r, w2e)
    as2v = h2t[:, _NCLS]
    ad2v = h2t[:, _NCLS + 1]

    part2 = _sc_layer2(srcd, dstd, h2t, as2v, ad2v)

    b2r = jnp.pad(b2, (0, _C2 - _NCLS)).reshape(1, _C2)
    o = _fin(part2, b2r)
    return o[:_N, :_NCLS]


# speedup vs baseline: 38.2483x; 1.1367x over previous
import functools

import jax
import jax.numpy as jnp
from jax import lax
import numpy as np
from jax.experimental import pallas as pl
from jax.experimental.pallas import tpu as pltpu
from jax.experimental.pallas import tpu_sc as plsc

_N = 10000
_E = 320000
_D = 128
_HID = 64
_HEADS = 8
_NCLS = 40

_NP = 10240
_NPT = _NP // 16
_K = 128
_K1 = 64
_EE = _E + _N
_EEP = 331776
_P1T = _EEP // 16
_P2T = _EEP // 32
_N1 = _P1T // _K
_N2 = _P2T // _K
_N11 = _P1T // _K1
_N21 = _P2T // _K1
_C2 = 48


def _leaky(x):
    return jnp.where(x >= 0.0, x, 0.2 * x)


_GDN = lax.GatherDimensionNumbers(offset_dims=(), collapsed_slice_dims=(0,),
                                  start_index_map=(0,))


def _vbcast(v, lane):
    idx = jnp.full((16, 1), lane, jnp.int32)
    return lax.gather(v, idx, _GDN, (1,),
                      mode=lax.GatherScatterMode.PROMISE_IN_BOUNDS)


def _mm1_body(x_ref, w_ref, asw_ref, adw_ref, h_ref, as_ref, ad_ref):
    h = jnp.dot(x_ref[...], w_ref[...], preferred_element_type=jnp.float32)
    h_ref[...] = h
    as_ref[...] = jnp.dot(h, asw_ref[...], preferred_element_type=jnp.float32)
    ad_ref[...] = jnp.dot(h, adw_ref[...], preferred_element_type=jnp.float32)


def _mm1(xp, W1, asw, adw, bm=1024):
    return pl.pallas_call(
        _mm1_body,
        grid=(_NP // bm,),
        in_specs=[pl.BlockSpec((bm, _D), lambda i: (i, 0)),
                  pl.BlockSpec((_D, 512), lambda i: (0, 0)),
                  pl.BlockSpec((512, 16), lambda i: (0, 0)),
                  pl.BlockSpec((512, 16), lambda i: (0, 0))],
        out_specs=[pl.BlockSpec((bm, 512), lambda i: (i, 0)),
                   pl.BlockSpec((bm, 16), lambda i: (i, 0)),
                   pl.BlockSpec((bm, 16), lambda i: (i, 0))],
        out_shape=[jax.ShapeDtypeStruct((_NP, 512), jnp.float32),
                   jax.ShapeDtypeStruct((_NP, 16), jnp.float32),
                   jax.ShapeDtypeStruct((_NP, 16), jnp.float32)],
    )(xp, W1, asw, adw)


def _mid_body(p0_ref, p1_ref, b1_ref, w2_ref, h2_ref):
    m = (p0_ref[...] + p1_ref[...]) * (1.0 / _HEADS) + b1_ref[...]
    h = jnp.where(m > 0.0, m, jnp.exp(jnp.minimum(m, 0.0)) - 1.0)
    h2_ref[...] = jnp.dot(h, w2_ref[...], preferred_element_type=jnp.float32)


def _mid(part, b1r, w2e, bm=1024):
    nb = _NP // bm
    return pl.pallas_call(
        _mid_body,
        grid=(nb,),
        in_specs=[pl.BlockSpec((bm, _HID), lambda i: (i, 0)),
                  pl.BlockSpec((bm, _HID), lambda i, _nb=nb: (i + _nb, 0)),
                  pl.BlockSpec((1, _HID), lambda i: (0, 0)),
                  pl.BlockSpec((_HID, _C2), lambda i: (0, 0))],
        out_specs=pl.BlockSpec((bm, _C2), lambda i: (i, 0)),
        out_shape=jax.ShapeDtypeStruct((_NP, _C2), jnp.float32),
    )(part, part, b1r, w2e)


def _fin_body(p0_ref, p1_ref, b2_ref, o_ref):
    x = p0_ref[...] + p1_ref[...] + b2_ref[...]
    col = lax.broadcasted_iota(jnp.int32, x.shape, 1)
    x = jnp.where(col < _NCLS, x, -1e30)
    m = jnp.max(x, axis=1, keepdims=True)
    lse = jnp.log(jnp.sum(jnp.exp(x - m), axis=1, keepdims=True))
    o_ref[...] = x - m - lse


def _fin(part2, b2r, bm=1024):
    nb = _NP // bm
    return pl.pallas_call(
        _fin_body,
        grid=(nb,),
        in_specs=[pl.BlockSpec((bm, _C2), lambda i: (i, 0)),
                  pl.BlockSpec((bm, _C2), lambda i, _nb=nb: (i + _nb, 0)),
                  pl.BlockSpec((1, _C2), lambda i: (0, 0))],
        out_specs=pl.BlockSpec((bm, _C2), lambda i: (i, 0)),
        out_shape=jax.ShapeDtypeStruct((_NP, _C2), jnp.float32),
    )(part2, part2, b2r)


def _l1_body(src_h, dst_h, ht_h, ast_h, adt_h,
             part_h, recip_h,
             idxs0, idxd0, idxs1, idxd1,
             ra0, rb0, rr0, ra1, rb1, rr1, h0, h1b, vout, strip,
             denom_sh, acc_sh, sem0, sem1):
    c = lax.axis_index("c")
    s = lax.axis_index("s")
    idxsl, idxdl = (idxs0, idxs1), (idxd0, idxd1)
    ral, rbl, rrl, hl = (ra0, ra1), (rb0, rb1), (rr0, rr1), (h0, h1b)
    seml = (sem0, sem1)

    def zs(i, _):
        strip[i, :] = jnp.zeros((16,), jnp.float32)
        return _
    lax.fori_loop(0, 128, zs, None)
    for kk in range(_NPT // 128):
        pltpu.sync_copy(strip, denom_sh.at[pl.ds(s * _NPT + kk * 128, 128)])

    def zv(i, _):
        vout[i // 4, pl.ds((i % 4) * 16, 16)] = jnp.zeros((16,), jnp.float32)
        return _
    lax.fori_loop(0, _K1 * 4, zv, None)
    for kk in range(_NPT // _K1):
        pltpu.sync_copy(vout, acc_sh.at[pl.ds(s * _NPT + kk * _K1, _K1)])
    plsc.subcore_barrier()

    def p1_start(bb, i):
        base = s * _P1T + i * _K1
        pltpu.async_copy(src_h.at[pl.ds(base, _K1)], idxsl[bb], seml[bb])
        pltpu.async_copy(dst_h.at[pl.ds(base, _K1)], idxdl[bb], seml[bb])
        pltpu.make_async_copy(src_h.at[pl.ds(base, _K1)], idxsl[bb], seml[bb]).wait()
        pltpu.make_async_copy(dst_h.at[pl.ds(base, _K1)], idxdl[bb], seml[bb]).wait()
        pltpu.async_copy(ast_h.at[idxsl[bb]], ral[bb], seml[bb])
        pltpu.async_copy(adt_h.at[idxdl[bb]], rbl[bb], seml[bb])

    def p1_fin(bb):
        pltpu.make_async_copy(ast_h.at[idxsl[bb]], ral[bb], seml[bb]).wait()
        pltpu.make_async_copy(adt_h.at[idxdl[bb]], rbl[bb], seml[bb]).wait()

        def cmp(e, _2):
            x = ral[bb][e, :] + rbl[bb][e, :]
            ral[bb][e, :] = jnp.exp(_leaky(x))
            return _2
        lax.fori_loop(0, _K1, cmp, None)
        pltpu.sync_copy(ral[bb], denom_sh.at[idxdl[bb]], add=True)

    p1_start(0, 0)

    def p1o(j, _):
        for bb in range(2):
            i = 2 * j + bb

            @pl.when(i + 1 < _N11)
            def _start():
                p1_start(1 - bb, i + 1)
            p1_fin(bb)
        return _
    lax.fori_loop(0, _N11 // 2, p1o, None)
    plsc.subcore_barrier()

    for kk in range(_NPT // 128):
        off = s * _NPT + kk * 128
        pltpu.sync_copy(denom_sh.at[pl.ds(off, 128)], strip)

        def rec(i, _):
            v = strip[i, :]
            strip[i, :] = 1.0 / (v + 1e-16)
            return _
        lax.fori_loop(0, 128, rec, None)
        pltpu.sync_copy(strip, recip_h.at[pl.ds(off, 128)])
    plsc.subcore_barrier()

    def p2_start(bb, i):
        base = c * (_EEP // 2) + s * _P2T + i * _K1
        pltpu.async_copy(src_h.at[pl.ds(base, _K1)], idxsl[bb], seml[bb])
        pltpu.async_copy(dst_h.at[pl.ds(base, _K1)], idxdl[bb], seml[bb])
        pltpu.make_async_copy(src_h.at[pl.ds(base, _K1)], idxsl[bb], seml[bb]).wait()
        pltpu.make_async_copy(dst_h.at[pl.ds(base, _K1)], idxdl[bb], seml[bb]).wait()
        pltpu.async_copy(ast_h.at[idxsl[bb]], ral[bb], seml[bb])
        pltpu.async_copy(adt_h.at[idxdl[bb]], rbl[bb], seml[bb])
        pltpu.async_copy(recip_h.at[idxdl[bb]], rrl[bb], seml[bb])
        pltpu.async_copy(ht_h.at[idxsl[bb]], hl[bb], seml[bb])

    def p2_fin(bb):
        pltpu.make_async_copy(ast_h.at[idxsl[bb]], ral[bb], seml[bb]).wait()
        pltpu.make_async_copy(adt_h.at[idxdl[bb]], rbl[bb], seml[bb]).wait()
        pltpu.make_async_copy(recip_h.at[idxdl[bb]], rrl[bb], seml[bb]).wait()
        pltpu.make_async_copy(ht_h.at[idxsl[bb]], hl[bb], seml[bb]).wait()

        def cmp(e, _2):
            x = ral[bb][e, :] + rbl[bb][e, :]
            cf = jnp.exp(_leaky(x)) * rrl[bb][e, :]
            accs = [jnp.zeros((16,), jnp.float32) for _j in range(4)]
            for h in range(_HEADS):
                bc = _vbcast(cf, h)
                for j in range(4):
                    accs[j] = accs[j] + bc * hl[bb][e, pl.ds(h * 64 + j * 16, 16)]
            for j in range(4):
                vout[e, pl.ds(j * 16, 16)] = accs[j]
            return _2
        lax.fori_loop(0, _K1, cmp, None)
        pltpu.sync_copy(vout, acc_sh.at[idxdl[bb]], add=True)

    p2_start(0, 0)

    def p2o(j, _):
        for bb in range(2):
            i = 2 * j + bb

            @pl.when(i + 1 < _N21)
            def _start():
                p2_start(1 - bb, i + 1)
            p2_fin(bb)
        return _
    lax.fori_loop(0, _N21 // 2, p2o, None)
    plsc.subcore_barrier()

    for kk in range(_NPT // _K1):
        pltpu.sync_copy(acc_sh.at[pl.ds(s * _NPT + kk * _K1, _K1)],
                        part_h.at[pl.ds(c * _NP + s * _NPT + kk * _K1, _K1)])


def _sc_layer1(srcd, dstd, ht, ast, adt):
    mesh = plsc.VectorSubcoreMesh(core_axis_name="c", subcore_axis_name="s")
    f = pl.kernel(
        _l1_body,
        out_type=[jax.ShapeDtypeStruct((2 * _NP, _HID), jnp.float32),
                  jax.ShapeDtypeStruct((_NP, 16), jnp.float32)],
        mesh=mesh,
        compiler_params=pltpu.CompilerParams(use_tc_tiling_on_sc=False, needs_layout_passes=False),
        scratch_types=[
            pltpu.VMEM((_K1,), jnp.int32),
            pltpu.VMEM((_K1,), jnp.int32),
            pltpu.VMEM((_K1,), jnp.int32),
            pltpu.VMEM((_K1,), jnp.int32),
            pltpu.VMEM((_K1, 16), jnp.float32),
            pltpu.VMEM((_K1, 16), jnp.float32),
            pltpu.VMEM((_K1, 16), jnp.float32),
            pltpu.VMEM((_K1, 16), jnp.float32),
            pltpu.VMEM((_K1, 16), jnp.float32),
            pltpu.VMEM((_K1, 16), jnp.float32),
            pltpu.VMEM((_K1, 512), jnp.float32),
            pltpu.VMEM((_K1, 512), jnp.float32),
            pltpu.VMEM((_K1, _HID), jnp.float32),
            pltpu.VMEM((128, 16), jnp.float32),
            pltpu.VMEM_SHARED((_NP, 16), jnp.float32),
            pltpu.VMEM_SHARED((_NP, _HID), jnp.float32),
            pltpu.SemaphoreType.DMA,
            pltpu.SemaphoreType.DMA,
        ],
    )
    return f(srcd, dstd, ht, ast, adt)


_K2 = 192
_N12 = _P1T // _K2
_N22 = _P2T // _K2


def _l2_body(src_h, dst_h, h2t_h, as2_h, ad2_h,
             part2_h,
             idxs0, idxd0, idxs1, idxd1, as2v, ad2v, rc2, exb, cfb,
             hr0, hr1, vout2, d2chunk,
             h2_sh, denom2_sh, acc2_sh, sem0, sem1):
    c = lax.axis_index("c")
    s = lax.axis_index("s")
    idxsl, idxdl = (idxs0, idxs1), (idxd0, idxd1)
    hrl, seml = (hr0, hr1), (sem0, sem1)

    pltpu.sync_copy(as2_h, as2v)
    pltpu.sync_copy(ad2_h, ad2v)
    pltpu.sync_copy(h2t_h.at[pl.ds(s * _NPT, _NPT)],
                    h2_sh.at[pl.ds(s * _NPT, _NPT)])

    def zd(i, _):
        d2chunk[pl.ds(i * 16, 16)] = jnp.zeros((16,), jnp.float32)
        return _
    lax.fori_loop(0, _NPT // 16, zd, None)
    pltpu.sync_copy(d2chunk, denom2_sh.at[pl.ds(s * _NPT, _NPT)])

    def zv(i, _):
        vout2[i // 3, pl.ds((i % 3) * 16, 16)] = jnp.zeros((16,), jnp.float32)
        return _
    lax.fori_loop(0, _K2 * 3, zv, None)
    for kk in range(0, _NPT, _K2):
        nrow = min(_K2, _NPT - kk)
        pltpu.sync_copy(vout2.at[pl.ds(0, nrow)],
                        acc2_sh.at[pl.ds(s * _NPT + kk, nrow)])
    plsc.subcore_barrier()

    def p1_start(bb, i):
        base = s * _P1T + i * _K2
        pltpu.async_copy(src_h.at[pl.ds(base, _K2)], idxsl[bb], seml[bb])
        pltpu.async_copy(dst_h.at[pl.ds(base, _K2)], idxdl[bb], seml[bb])

    def p1_fin(bb):
        pltpu.make_async_copy(src_h.at[pl.ds(0, _K2)], idxsl[bb], seml[bb]).wait()
        pltpu.make_async_copy(dst_h.at[pl.ds(0, _K2)], idxdl[bb], seml[bb]).wait()

        def cmp(t, _2):
            sv = idxsl[bb][pl.ds(t * 16, 16)]
            dv = idxdl[bb][pl.ds(t * 16, 16)]
            aa = plsc.load_gather(as2v, [sv])
            ab = plsc.load_gather(ad2v, [dv])
            exb[pl.ds(t * 16, 16)] = jnp.exp(_leaky(aa + ab))
            return _2
        lax.fori_loop(0, _K2 // 16, cmp, None)
        pltpu.sync_copy(exb, denom2_sh.at[idxdl[bb]], add=True)

    p1_start(0, 0)

    def p1o(j, _):
        for bb in range(2):
            i = 2 * j + bb

            @pl.when(i + 1 < _N12)
            def _st():
                p1_start(1 - bb, i + 1)
            p1_fin(bb)
        return _
    lax.fori_loop(0, _N12 // 2, p1o, None)
    plsc.subcore_barrier()

    pltpu.sync_copy(denom2_sh.at[pl.ds(s * _NPT, _NPT)], d2chunk)

    def rec(i, _):
        v = d2chunk[pl.ds(i * 16, 16)]
        d2chunk[pl.ds(i * 16, 16)] = 1.0 / (v + 1e-16)
        return _
    lax.fori_loop(0, _NPT // 16, rec, None)
    pltpu.sync_copy(d2chunk, denom2_sh.at[pl.ds(s * _NPT, _NPT)])
    plsc.subcore_barrier()
    pltpu.sync_copy(denom2_sh, rc2)

    def p2_start(bb, i):
        base = c * (_EEP // 2) + s * _P2T + i * _K2
        pltpu.sync_copy(src_h.at[pl.ds(base, _K2)], idxsl[bb])
        pltpu.sync_copy(dst_h.at[pl.ds(base, _K2)], idxdl[bb])
        pltpu.async_copy(h2_sh.at[idxsl[bb]], hrl[bb], seml[bb])

    def p2_fin(bb):
        pltpu.make_async_copy(h2_sh.at[idxsl[bb]], hrl[bb], seml[bb]).wait()

        def cmp(t, _2):
            sv = idxsl[bb][pl.ds(t * 16, 16)]
            dv = idxdl[bb][pl.ds(t * 16, 16)]
            aa = plsc.load_gather(as2v, [sv])
            ab = plsc.load_gather(ad2v, [dv])
            r = plsc.load_gather(rc2, [dv])
            cfb[pl.ds(t * 16, 16)] = jnp.exp(_leaky(aa + ab)) * r
            return _2
        lax.fori_loop(0, _K2 // 16, cmp, None)

        def rowm(e, _2):
            bc = _vbcast(cfb[pl.ds((e // 16) * 16, 16)], e % 16)
            for j in range(3):
                vout2[e, pl.ds(j * 16, 16)] = bc * hrl[bb][e, pl.ds(j * 16, 16)]
            return _2
        lax.fori_loop(0, _K2, rowm, None)
        pltpu.sync_copy(vout2, acc2_sh.at[idxdl[bb]], add=True)

    p2_start(0, 0)

    def p2o(j, _):
        for bb in range(2):
            i = 2 * j + bb

            @pl.when(i + 1 < _N22)
            def _st():
                p2_start(1 - bb, i + 1)
            p2_fin(bb)
        return _
    lax.fori_loop(0, _N22 // 2, p2o, None)
    plsc.subcore_barrier()

    for kk in range(0, _NPT, _K2):
        nrow = min(_K2, _NPT - kk)
        pltpu.sync_copy(acc2_sh.at[pl.ds(s * _NPT + kk, nrow)],
                        part2_h.at[pl.ds(c * _NP + s * _NPT + kk, nrow)])


def _sc_layer2(srcd, dstd, h2t, as2v, ad2v):
    mesh = plsc.VectorSubcoreMesh(core_axis_name="c", subcore_axis_name="s")
    f = pl.kernel(
        _l2_body,
        out_type=jax.ShapeDtypeStruct((2 * _NP, _C2), jnp.float32),
        mesh=mesh,
        compiler_params=pltpu.CompilerParams(use_tc_tiling_on_sc=False, needs_layout_passes=False),
        scratch_types=[
            pltpu.VMEM((_K2,), jnp.int32),
            pltpu.VMEM((_K2,), jnp.int32),
            pltpu.VMEM((_K2,), jnp.int32),
            pltpu.VMEM((_K2,), jnp.int32),
            pltpu.VMEM((_NP,), jnp.float32),
            pltpu.VMEM((_NP,), jnp.float32),
            pltpu.VMEM((_NP,), jnp.float32),
            pltpu.VMEM((_K2,), jnp.float32),
            pltpu.VMEM((_K2,), jnp.float32),
            pltpu.VMEM((_K2, _C2), jnp.float32),
            pltpu.VMEM((_K2, _C2), jnp.float32),
            pltpu.VMEM((_K2, _C2), jnp.float32),
            pltpu.VMEM((_NPT,), jnp.float32),
            pltpu.VMEM_SHARED((_NP, _C2), jnp.float32),
            pltpu.VMEM_SHARED((_NP,), jnp.float32),
            pltpu.VMEM_SHARED((_NP, _C2), jnp.float32),
            pltpu.SemaphoreType.DMA,
            pltpu.SemaphoreType.DMA,
        ],
    )
    return f(srcd, dstd, h2t, as2v, ad2v)


def kernel(x, edge_index, W1, a_src1, a_dst1, b1, W2, a_src2, a_dst2, b2):
    sl = jnp.arange(_N, dtype=jnp.int32)
    npad = _EEP - _EE
    padidx = _N + (jnp.arange(npad, dtype=jnp.int32) % 240)
    srcd = jnp.concatenate([edge_index[0].astype(jnp.int32), sl, padidx])
    dstd = jnp.concatenate([edge_index[1].astype(jnp.int32), sl, padidx])

    rows = jnp.arange(512, dtype=jnp.int32)
    hcol = rows // _HID
    asw = jnp.zeros((512, 16), jnp.float32).at[rows, hcol].set(
        a_src1.reshape(512))
    adw = jnp.zeros((512, 16), jnp.float32).at[rows, hcol].set(
        a_dst1.reshape(512))

    xp = jnp.pad(x, ((0, _NP - _N), (0, 0)))
    ht, ast, adt = _mm1(xp, W1, asw, adw)

    part, _recip = _sc_layer1(srcd, dstd, ht, ast, adt)

    w_as2 = (W2 @ a_src2[0]).reshape(_HID, 1)
    w_ad2 = (W2 @ a_dst2[0]).reshape(_HID, 1)
    w2e = jnp.concatenate(
        [W2, w_as2, w_ad2, jnp.zeros((_HID, _C2 - _NCLS - 2), jnp.float32)],
        axis=1)
    b1r = b1.reshape(1, _HID)
    h2t = _mid(part, b1r, w2e)
    as2v = h2t[:, _NCLS]
    ad2v = h2t[:, _NCLS + 1]

    part2 = _sc_layer2(srcd, dstd, h2t, as2v, ad2v)

    b2r = jnp.pad(b2, (0, _C2 - _NCLS)).reshape(1, _C2)
    o = _fin(part2, b2r)
    return o[:_N, :_NCLS]


# speedup vs baseline: 44.7745x; 1.1706x over previous
import functools

import jax
import jax.numpy as jnp
from jax import lax
import numpy as np
from jax.experimental import pallas as pl
from jax.experimental.pallas import tpu as pltpu
from jax.experimental.pallas import tpu_sc as plsc

_N = 10000
_E = 320000
_D = 128
_HID = 64
_HEADS = 8
_NCLS = 40

_NP = 10240
_NPT = _NP // 16
_K = 128
_K1 = 64
_EE = _E + _N
_EEP = 331776
_P1T = _EEP // 16
_P2T = _EEP // 32
_N1 = _P1T // _K
_N2 = _P2T // _K
_N11 = _P1T // _K1
_N21 = _P2T // _K1
_C2 = 48


def _leaky(x):
    return jnp.where(x >= 0.0, x, 0.2 * x)


_GDN = lax.GatherDimensionNumbers(offset_dims=(), collapsed_slice_dims=(0,),
                                  start_index_map=(0,))


def _vbcast(v, lane):
    idx = jnp.full((16, 1), lane, jnp.int32)
    return lax.gather(v, idx, _GDN, (1,),
                      mode=lax.GatherScatterMode.PROMISE_IN_BOUNDS)


def _mm1_body(x_ref, w_ref, asw_ref, adw_ref, h_ref, as_ref, ad_ref):
    h = jnp.dot(x_ref[...], w_ref[...], preferred_element_type=jnp.float32)
    h_ref[...] = h
    as_ref[...] = jnp.dot(h, asw_ref[...], preferred_element_type=jnp.float32)
    ad_ref[...] = jnp.dot(h, adw_ref[...], preferred_element_type=jnp.float32)


def _mm1(xp, W1, asw, adw, bm=1024):
    return pl.pallas_call(
        _mm1_body,
        grid=(_NP // bm,),
        in_specs=[pl.BlockSpec((bm, _D), lambda i: (i, 0)),
                  pl.BlockSpec((_D, 512), lambda i: (0, 0)),
                  pl.BlockSpec((512, 16), lambda i: (0, 0)),
                  pl.BlockSpec((512, 16), lambda i: (0, 0))],
        out_specs=[pl.BlockSpec((bm, 512), lambda i: (i, 0)),
                   pl.BlockSpec((bm, 16), lambda i: (i, 0)),
                   pl.BlockSpec((bm, 16), lambda i: (i, 0))],
        out_shape=[jax.ShapeDtypeStruct((_NP, 512), jnp.float32),
                   jax.ShapeDtypeStruct((_NP, 16), jnp.float32),
                   jax.ShapeDtypeStruct((_NP, 16), jnp.float32)],
    )(xp, W1, asw, adw)


def _mid_body(p0_ref, p1_ref, b1_ref, w2_ref, h2_ref):
    m = (p0_ref[...] + p1_ref[...]) * (1.0 / _HEADS) + b1_ref[...]
    h = jnp.where(m > 0.0, m, jnp.exp(jnp.minimum(m, 0.0)) - 1.0)
    h2_ref[...] = jnp.dot(h, w2_ref[...], preferred_element_type=jnp.float32)


def _mid(part, b1r, w2e, bm=1024):
    nb = _NP // bm
    return pl.pallas_call(
        _mid_body,
        grid=(nb,),
        in_specs=[pl.BlockSpec((bm, _HID), lambda i: (i, 0)),
                  pl.BlockSpec((bm, _HID), lambda i, _nb=nb: (i + _nb, 0)),
                  pl.BlockSpec((1, _HID), lambda i: (0, 0)),
                  pl.BlockSpec((_HID, _C2), lambda i: (0, 0))],
        out_specs=pl.BlockSpec((bm, _C2), lambda i: (i, 0)),
        out_shape=jax.ShapeDtypeStruct((_NP, _C2), jnp.float32),
    )(part, part, b1r, w2e)


def _fin_body(p0_ref, p1_ref, b2_ref, o_ref):
    x = p0_ref[...] + p1_ref[...] + b2_ref[...]
    col = lax.broadcasted_iota(jnp.int32, x.shape, 1)
    x = jnp.where(col < _NCLS, x, -1e30)
    m = jnp.max(x, axis=1, keepdims=True)
    lse = jnp.log(jnp.sum(jnp.exp(x - m), axis=1, keepdims=True))
    o_ref[...] = x - m - lse


def _fin(part2, b2r, bm=1024):
    nb = _NP // bm
    return pl.pallas_call(
        _fin_body,
        grid=(nb,),
        in_specs=[pl.BlockSpec((bm, _C2), lambda i: (i, 0)),
                  pl.BlockSpec((bm, _C2), lambda i, _nb=nb: (i + _nb, 0)),
                  pl.BlockSpec((1, _C2), lambda i: (0, 0))],
        out_specs=pl.BlockSpec((bm, _C2), lambda i: (i, 0)),
        out_shape=jax.ShapeDtypeStruct((_NP, _C2), jnp.float32),
    )(part2, part2, b2r)


def _l1_body(src_h, dst_h, ht_h, ast_h, adt_h,
             part_h, recip_h,
             idxs0, idxd0, idxs1, idxd1, idxs2, idxd2,
             ra0, rb0, rr0, ra1, rb1, rr1, h0, h1b, vout, strip,
             denom_sh, acc_sh, sem0, sem1, semi0, semi1, semi2):
    c = lax.axis_index("c")
    s = lax.axis_index("s")
    idxsl, idxdl = (idxs0, idxs1, idxs2), (idxd0, idxd1, idxd2)
    ral, rbl, rrl, hl = (ra0, ra1), (rb0, rb1), (rr0, rr1), (h0, h1b)
    seml = (sem0, sem1)
    semil = (semi0, semi1, semi2)

    def zs(i, _):
        strip[i, :] = jnp.zeros((16,), jnp.float32)
        return _
    lax.fori_loop(0, 128, zs, None)
    for kk in range(_NPT // 128):
        pltpu.sync_copy(strip, denom_sh.at[pl.ds(s * _NPT + kk * 128, 128)])

    def zv(i, _):
        vout[i // 4, pl.ds((i % 4) * 16, 16)] = jnp.zeros((16,), jnp.float32)
        return _
    lax.fori_loop(0, _K1 * 4, zv, None)
    for kk in range(_NPT // _K1):
        pltpu.sync_copy(vout, acc_sh.at[pl.ds(s * _NPT + kk * _K1, _K1)])
    plsc.subcore_barrier()

    def idx_start(sl, i, ebase):
        base = ebase + i * _K1
        pltpu.async_copy(src_h.at[pl.ds(base, _K1)], idxsl[sl], semil[sl])
        pltpu.async_copy(dst_h.at[pl.ds(base, _K1)], idxdl[sl], semil[sl])

    def idx_wait(sl):
        pltpu.make_async_copy(src_h.at[pl.ds(0, _K1)], idxsl[sl], semil[sl]).wait()
        pltpu.make_async_copy(dst_h.at[pl.ds(0, _K1)], idxdl[sl], semil[sl]).wait()

    def p1_gather(sl, bb):
        pltpu.async_copy(ast_h.at[idxsl[sl]], ral[bb], seml[bb])
        pltpu.async_copy(adt_h.at[idxdl[sl]], rbl[bb], seml[bb])

    def p1_fin(sl, bb):
        pltpu.make_async_copy(ast_h.at[idxsl[sl]], ral[bb], seml[bb]).wait()
        pltpu.make_async_copy(adt_h.at[idxdl[sl]], rbl[bb], seml[bb]).wait()

        def cmp(e, _2):
            x = ral[bb][e, :] + rbl[bb][e, :]
            ral[bb][e, :] = jnp.exp(_leaky(x))
            return _2
        lax.fori_loop(0, _K1, cmp, None)
        pltpu.sync_copy(ral[bb], denom_sh.at[idxdl[sl]], add=True)

    eb1 = s * _P1T
    idx_start(0, 0, eb1)
    idx_start(1, 1, eb1)
    idx_wait(0)
    p1_gather(0, 0)

    def p1o(j, _):
        for u in range(6):
            i = 6 * j + u

            @pl.when(i + 2 < _N11)
            def _sti():
                idx_start((u + 2) % 3, i + 2, eb1)

            @pl.when(i + 1 < _N11)
            def _stg():
                idx_wait((u + 1) % 3)
                p1_gather((u + 1) % 3, (u + 1) % 2)
            p1_fin(u % 3, u % 2)
        return _
    lax.fori_loop(0, _N11 // 6, p1o, None)
    plsc.subcore_barrier()

    for kk in range(_NPT // 128):
        off = s * _NPT + kk * 128
        pltpu.sync_copy(denom_sh.at[pl.ds(off, 128)], strip)

        def rec(i, _):
            v = strip[i, :]
            strip[i, :] = 1.0 / (v + 1e-16)
            return _
        lax.fori_loop(0, 128, rec, None)
        pltpu.sync_copy(strip, recip_h.at[pl.ds(off, 128)])
    plsc.subcore_barrier()

    def p2_gather(sl, bb):
        pltpu.async_copy(ast_h.at[idxsl[sl]], ral[bb], seml[bb])
        pltpu.async_copy(adt_h.at[idxdl[sl]], rbl[bb], seml[bb])
        pltpu.async_copy(recip_h.at[idxdl[sl]], rrl[bb], seml[bb])
        pltpu.async_copy(ht_h.at[idxsl[sl]], hl[bb], seml[bb])

    def p2_fin(sl, bb):
        pltpu.make_async_copy(ast_h.at[idxsl[sl]], ral[bb], seml[bb]).wait()
        pltpu.make_async_copy(adt_h.at[idxdl[sl]], rbl[bb], seml[bb]).wait()
        pltpu.make_async_copy(recip_h.at[idxdl[sl]], rrl[bb], seml[bb]).wait()
        pltpu.make_async_copy(ht_h.at[idxsl[sl]], hl[bb], seml[bb]).wait()

        def cmp(e, _2):
            x = ral[bb][e, :] + rbl[bb][e, :]
            cf = jnp.exp(_leaky(x)) * rrl[bb][e, :]
            accs = [jnp.zeros((16,), jnp.float32) for _j in range(4)]
            for h in range(_HEADS):
                bc = _vbcast(cf, h)
                for j in range(4):
                    accs[j] = accs[j] + bc * hl[bb][e, pl.ds(h * 64 + j * 16, 16)]
            for j in range(4):
                vout[e, pl.ds(j * 16, 16)] = accs[j]
            return _2
        lax.fori_loop(0, _K1, cmp, None)
        pltpu.sync_copy(vout, acc_sh.at[idxdl[sl]], add=True)

    eb2 = c * (_EEP // 2) + s * _P2T
    idx_start(0, 0, eb2)
    idx_start(1, 1, eb2)
    idx_wait(0)
    p2_gather(0, 0)

    def p2o(j, _):
        for u in range(6):
            i = 6 * j + u

            @pl.when(i + 2 < _N21)
            def _sti():
                idx_start((u + 2) % 3, i + 2, eb2)

            @pl.when(i + 1 < _N21)
            def _stg():
                idx_wait((u + 1) % 3)
                p2_gather((u + 1) % 3, (u + 1) % 2)
            p2_fin(u % 3, u % 2)
        return _
    lax.fori_loop(0, _N21 // 6, p2o, None)
    plsc.subcore_barrier()

    for kk in range(_NPT // _K1):
        pltpu.sync_copy(acc_sh.at[pl.ds(s * _NPT + kk * _K1, _K1)],
                        part_h.at[pl.ds(c * _NP + s * _NPT + kk * _K1, _K1)])


def _sc_layer1(srcd, dstd, ht, ast, adt):
    mesh = plsc.VectorSubcoreMesh(core_axis_name="c", subcore_axis_name="s")
    f = pl.kernel(
        _l1_body,
        out_type=[jax.ShapeDtypeStruct((2 * _NP, _HID), jnp.float32),
                  jax.ShapeDtypeStruct((_NP, 16), jnp.float32)],
        mesh=mesh,
        compiler_params=pltpu.CompilerParams(use_tc_tiling_on_sc=False, needs_layout_passes=False),
        scratch_types=[
            pltpu.VMEM((_K1,), jnp.int32),
            pltpu.VMEM((_K1,), jnp.int32),
            pltpu.VMEM((_K1,), jnp.int32),
            pltpu.VMEM((_K1,), jnp.int32),
            pltpu.VMEM((_K1,), jnp.int32),
            pltpu.VMEM((_K1,), jnp.int32),
            pltpu.VMEM((_K1, 16), jnp.float32),
            pltpu.VMEM((_K1, 16), jnp.float32),
            pltpu.VMEM((_K1, 16), jnp.float32),
            pltpu.VMEM((_K1, 16), jnp.float32),
            pltpu.VMEM((_K1, 16), jnp.float32),
            pltpu.VMEM((_K1, 16), jnp.float32),
            pltpu.VMEM((_K1, 512), jnp.float32),
            pltpu.VMEM((_K1, 512), jnp.float32),
            pltpu.VMEM((_K1, _HID), jnp.float32),
            pltpu.VMEM((128, 16), jnp.float32),
            pltpu.VMEM_SHARED((_NP, 16), jnp.float32),
            pltpu.VMEM_SHARED((_NP, _HID), jnp.float32),
            pltpu.SemaphoreType.DMA,
            pltpu.SemaphoreType.DMA,
            pltpu.SemaphoreType.DMA,
            pltpu.SemaphoreType.DMA,
            pltpu.SemaphoreType.DMA,
        ],
    )
    return f(srcd, dstd, ht, ast, adt)


_K2 = 192
_N12 = _P1T // _K2
_N22 = _P2T // _K2


def _l2_body(src_h, dst_h, h2t_h, as2_h, ad2_h,
             part2_h,
             idxs0, idxd0, idxs1, idxd1, as2v, ad2v, rc2, exb, cfb,
             hr0, hr1, vout2, d2chunk,
             h2_sh, denom2_sh, acc2_sh, sem0, sem1):
    c = lax.axis_index("c")
    s = lax.axis_index("s")
    idxsl, idxdl = (idxs0, idxs1), (idxd0, idxd1)
    hrl, seml = (hr0, hr1), (sem0, sem1)

    pltpu.sync_copy(as2_h, as2v)
    pltpu.sync_copy(ad2_h, ad2v)
    pltpu.sync_copy(h2t_h.at[pl.ds(s * _NPT, _NPT)],
                    h2_sh.at[pl.ds(s * _NPT, _NPT)])

    def zd(i, _):
        d2chunk[pl.ds(i * 16, 16)] = jnp.zeros((16,), jnp.float32)
        return _
    lax.fori_loop(0, _NPT // 16, zd, None)
    pltpu.sync_copy(d2chunk, denom2_sh.at[pl.ds(s * _NPT, _NPT)])

    def zv(i, _):
        vout2[i // 3, pl.ds((i % 3) * 16, 16)] = jnp.zeros((16,), jnp.float32)
        return _
    lax.fori_loop(0, _K2 * 3, zv, None)
    for kk in range(0, _NPT, _K2):
        nrow = min(_K2, _NPT - kk)
        pltpu.sync_copy(vout2.at[pl.ds(0, nrow)],
                        acc2_sh.at[pl.ds(s * _NPT + kk, nrow)])
    plsc.subcore_barrier()

    def p1_start(bb, i):
        base = s * _P1T + i * _K2
        pltpu.async_copy(src_h.at[pl.ds(base, _K2)], idxsl[bb], seml[bb])
        pltpu.async_copy(dst_h.at[pl.ds(base, _K2)], idxdl[bb], seml[bb])

    def p1_fin(bb):
        pltpu.make_async_copy(src_h.at[pl.ds(0, _K2)], idxsl[bb], seml[bb]).wait()
        pltpu.make_async_copy(dst_h.at[pl.ds(0, _K2)], idxdl[bb], seml[bb]).wait()

        def cmp(t, _2):
            sv = idxsl[bb][pl.ds(t * 16, 16)]
            dv = idxdl[bb][pl.ds(t * 16, 16)]
            aa = plsc.load_gather(as2v, [sv])
            ab = plsc.load_gather(ad2v, [dv])
            exb[pl.ds(t * 16, 16)] = jnp.exp(_leaky(aa + ab))
            return _2
        lax.fori_loop(0, _K2 // 16, cmp, None)
        pltpu.sync_copy(exb, denom2_sh.at[idxdl[bb]], add=True)

    p1_start(0, 0)

    def p1o(j, _):
        for bb in range(2):
            i = 2 * j + bb

            @pl.when(i + 1 < _N12)
            def _st():
                p1_start(1 - bb, i + 1)
            p1_fin(bb)
        return _
    lax.fori_loop(0, _N12 // 2, p1o, None)
    plsc.subcore_barrier()

    pltpu.sync_copy(denom2_sh.at[pl.ds(s * _NPT, _NPT)], d2chunk)

    def rec(i, _):
        v = d2chunk[pl.ds(i * 16, 16)]
        d2chunk[pl.ds(i * 16, 16)] = 1.0 / (v + 1e-16)
        return _
    lax.fori_loop(0, _NPT // 16, rec, None)
    pltpu.sync_copy(d2chunk, denom2_sh.at[pl.ds(s * _NPT, _NPT)])
    plsc.subcore_barrier()
    pltpu.sync_copy(denom2_sh, rc2)

    def p2_start(bb, i):
        base = c * (_EEP // 2) + s * _P2T + i * _K2
        pltpu.sync_copy(src_h.at[pl.ds(base, _K2)], idxsl[bb])
        pltpu.sync_copy(dst_h.at[pl.ds(base, _K2)], idxdl[bb])
        pltpu.async_copy(h2_sh.at[idxsl[bb]], hrl[bb], seml[bb])

    def p2_fin(bb):
        pltpu.make_async_copy(h2_sh.at[idxsl[bb]], hrl[bb], seml[bb]).wait()

        def cmp(t, _2):
            sv = idxsl[bb][pl.ds(t * 16, 16)]
            dv = idxdl[bb][pl.ds(t * 16, 16)]
            aa = plsc.load_gather(as2v, [sv])
            ab = plsc.load_gather(ad2v, [dv])
            r = plsc.load_gather(rc2, [dv])
            cfb[pl.ds(t * 16, 16)] = jnp.exp(_leaky(aa + ab)) * r
            return _2
        lax.fori_loop(0, _K2 // 16, cmp, None)

        def rowm(e, _2):
            bc = _vbcast(cfb[pl.ds((e // 16) * 16, 16)], e % 16)
            for j in range(3):
                vout2[e, pl.ds(j * 16, 16)] = bc * hrl[bb][e, pl.ds(j * 16, 16)]
            return _2
        lax.fori_loop(0, _K2, rowm, None)
        pltpu.sync_copy(vout2, acc2_sh.at[idxdl[bb]], add=True)

    p2_start(0, 0)

    def p2o(j, _):
        for bb in range(2):
            i = 2 * j + bb

            @pl.when(i + 1 < _N22)
            def _st():
                p2_start(1 - bb, i + 1)
            p2_fin(bb)
        return _
    lax.fori_loop(0, _N22 // 2, p2o, None)
    plsc.subcore_barrier()

    for kk in range(0, _NPT, _K2):
        nrow = min(_K2, _NPT - kk)
        pltpu.sync_copy(acc2_sh.at[pl.ds(s * _NPT + kk, nrow)],
                        part2_h.at[pl.ds(c * _NP + s * _NPT + kk, nrow)])


def _sc_layer2(srcd, dstd, h2t, as2v, ad2v):
    mesh = plsc.VectorSubcoreMesh(core_axis_name="c", subcore_axis_name="s")
    f = pl.kernel(
        _l2_body,
        out_type=jax.ShapeDtypeStruct((2 * _NP, _C2), jnp.float32),
        mesh=mesh,
        compiler_params=pltpu.CompilerParams(use_tc_tiling_on_sc=False, needs_layout_passes=False),
        scratch_types=[
            pltpu.VMEM((_K2,), jnp.int32),
            pltpu.VMEM((_K2,), jnp.int32),
            pltpu.VMEM((_K2,), jnp.int32),
            pltpu.VMEM((_K2,), jnp.int32),
            pltpu.VMEM((_NP,), jnp.float32),
            pltpu.VMEM((_NP,), jnp.float32),
            pltpu.VMEM((_NP,), jnp.float32),
            pltpu.VMEM((_K2,), jnp.float32),
            pltpu.VMEM((_K2,), jnp.float32),
            pltpu.VMEM((_K2, _C2), jnp.float32),
            pltpu.VMEM((_K2, _C2), jnp.float32),
            pltpu.VMEM((_K2, _C2), jnp.float32),
            pltpu.VMEM((_NPT,), jnp.float32),
            pltpu.VMEM_SHARED((_NP, _C2), jnp.float32),
            pltpu.VMEM_SHARED((_NP,), jnp.float32),
            pltpu.VMEM_SHARED((_NP, _C2), jnp.float32),
            pltpu.SemaphoreType.DMA,
            pltpu.SemaphoreType.DMA,
        ],
    )
    return f(srcd, dstd, h2t, as2v, ad2v)


def kernel(x, edge_index, W1, a_src1, a_dst1, b1, W2, a_src2, a_dst2, b2):
    sl = jnp.arange(_N, dtype=jnp.int32)
    npad = _EEP - _EE
    padidx = _N + (jnp.arange(npad, dtype=jnp.int32) % 240)
    srcd = jnp.concatenate([edge_index[0].astype(jnp.int32), sl, padidx])
    dstd = jnp.concatenate([edge_index[1].astype(jnp.int32), sl, padidx])

    rows = jnp.arange(512, dtype=jnp.int32)
    hcol = rows // _HID
    asw = jnp.zeros((512, 16), jnp.float32).at[rows, hcol].set(
        a_src1.reshape(512))
    adw = jnp.zeros((512, 16), jnp.float32).at[rows, hcol].set(
        a_dst1.reshape(512))

    xp = jnp.pad(x, ((0, _NP - _N), (0, 0)))
    ht, ast, adt = _mm1(xp, W1, asw, adw)

    part, _recip = _sc_layer1(srcd, dstd, ht, ast, adt)

    w_as2 = (W2 @ a_src2[0]).reshape(_HID, 1)
    w_ad2 = (W2 @ a_dst2[0]).reshape(_HID, 1)
    w2e = jnp.concatenate(
        [W2, w_as2, w_ad2, jnp.zeros((_HID, _C2 - _NCLS - 2), jnp.float32)],
        axis=1)
    b1r = b1.reshape(1, _HID)
    h2t = _mid(part, b1r, w2e)
    as2v = h2t[:, _NCLS]
    ad2v = h2t[:, _NCLS + 1]

    part2 = _sc_layer2(srcd, dstd, h2t, as2v, ad2v)

    b2r = jnp.pad(b2, (0, _C2 - _NCLS)).reshape(1, _C2)
    o = _fin(part2, b2r)
    return o[:_N, :_NCLS]


# speedup vs baseline: 46.6398x; 1.0417x over previous
import functools

import jax
import jax.numpy as jnp
from jax import lax
import numpy as np
from jax.experimental import pallas as pl
from jax.experimental.pallas import tpu as pltpu
from jax.experimental.pallas import tpu_sc as plsc

_N = 10000
_E = 320000
_D = 128
_HID = 64
_HEADS = 8
_NCLS = 40

_NP = 10240
_NPT = _NP // 16
_K = 128
_K1 = 64
_EE = _E + _N
_EEP = 331776
_P1T = _EEP // 16
_P2T = _EEP // 32
_N1 = _P1T // _K
_N2 = _P2T // _K
_N11 = _P1T // _K1
_N21 = _P2T // _K1
_C2 = 48


def _leaky(x):
    return jnp.where(x >= 0.0, x, 0.2 * x)


_GDN = lax.GatherDimensionNumbers(offset_dims=(), collapsed_slice_dims=(0,),
                                  start_index_map=(0,))


def _vbcast(v, lane):
    idx = jnp.full((16, 1), lane, jnp.int32)
    return lax.gather(v, idx, _GDN, (1,),
                      mode=lax.GatherScatterMode.PROMISE_IN_BOUNDS)


def _mm1_body(x_ref, w_ref, asw_ref, adw_ref, h_ref, as_ref, ad_ref):
    h = jnp.dot(x_ref[...], w_ref[...], preferred_element_type=jnp.float32)
    h_ref[...] = h
    as_ref[...] = jnp.dot(h, asw_ref[...], preferred_element_type=jnp.float32)
    ad_ref[...] = jnp.dot(h, adw_ref[...], preferred_element_type=jnp.float32)


def _mm1(xp, W1, asw, adw, bm=1024):
    return pl.pallas_call(
        _mm1_body,
        grid=(_NP // bm,),
        in_specs=[pl.BlockSpec((bm, _D), lambda i: (i, 0)),
                  pl.BlockSpec((_D, 512), lambda i: (0, 0)),
                  pl.BlockSpec((512, 16), lambda i: (0, 0)),
                  pl.BlockSpec((512, 16), lambda i: (0, 0))],
        out_specs=[pl.BlockSpec((bm, 512), lambda i: (i, 0)),
                   pl.BlockSpec((bm, 16), lambda i: (i, 0)),
                   pl.BlockSpec((bm, 16), lambda i: (i, 0))],
        out_shape=[jax.ShapeDtypeStruct((_NP, 512), jnp.float32),
                   jax.ShapeDtypeStruct((_NP, 16), jnp.float32),
                   jax.ShapeDtypeStruct((_NP, 16), jnp.float32)],
    )(xp, W1, asw, adw)


def _mid_body(p0_ref, p1_ref, b1_ref, w2_ref, h2_ref):
    m = (p0_ref[...] + p1_ref[...]) * (1.0 / _HEADS) + b1_ref[...]
    h = jnp.where(m > 0.0, m, jnp.exp(jnp.minimum(m, 0.0)) - 1.0)
    h2_ref[...] = jnp.dot(h, w2_ref[...], preferred_element_type=jnp.float32)


def _mid(part, b1r, w2e, bm=1024):
    nb = _NP // bm
    return pl.pallas_call(
        _mid_body,
        grid=(nb,),
        in_specs=[pl.BlockSpec((bm, _HID), lambda i: (i, 0)),
                  pl.BlockSpec((bm, _HID), lambda i, _nb=nb: (i + _nb, 0)),
                  pl.BlockSpec((1, _HID), lambda i: (0, 0)),
                  pl.BlockSpec((_HID, _C2), lambda i: (0, 0))],
        out_specs=pl.BlockSpec((bm, _C2), lambda i: (i, 0)),
        out_shape=jax.ShapeDtypeStruct((_NP, _C2), jnp.float32),
    )(part, part, b1r, w2e)


def _fin_body(p0_ref, p1_ref, b2_ref, o_ref):
    x = p0_ref[...] + p1_ref[...] + b2_ref[...]
    col = lax.broadcasted_iota(jnp.int32, x.shape, 1)
    x = jnp.where(col < _NCLS, x, -1e30)
    m = jnp.max(x, axis=1, keepdims=True)
    lse = jnp.log(jnp.sum(jnp.exp(x - m), axis=1, keepdims=True))
    o_ref[...] = x - m - lse


def _fin(part2, b2r, bm=1024):
    nb = _NP // bm
    return pl.pallas_call(
        _fin_body,
        grid=(nb,),
        in_specs=[pl.BlockSpec((bm, _C2), lambda i: (i, 0)),
                  pl.BlockSpec((bm, _C2), lambda i, _nb=nb: (i + _nb, 0)),
                  pl.BlockSpec((1, _C2), lambda i: (0, 0))],
        out_specs=pl.BlockSpec((bm, _C2), lambda i: (i, 0)),
        out_shape=jax.ShapeDtypeStruct((_NP, _C2), jnp.float32),
    )(part2, part2, b2r)


def _l1_body(src_h, dst_h, ht_h, ast_h, adt_h,
             part_h, recip_h,
             idxs0, idxd0, idxs1, idxd1, idxs2, idxd2,
             ra0, rb0, rr0, ra1, rb1, rr1, h0, h1b, vout, strip,
             denom_sh, acc_sh, sem0, sem1, semi0, semi1, semi2):
    c = lax.axis_index("c")
    s = lax.axis_index("s")
    idxsl, idxdl = (idxs0, idxs1, idxs2), (idxd0, idxd1, idxd2)
    ral, rbl, rrl, hl = (ra0, ra1), (rb0, rb1), (rr0, rr1), (h0, h1b)
    seml = (sem0, sem1)
    semil = (semi0, semi1, semi2)

    def zs(i, _):
        strip[i, :] = jnp.zeros((16,), jnp.float32)
        return _
    lax.fori_loop(0, 128, zs, None)
    for kk in range(_NPT // 128):
        pltpu.sync_copy(strip, denom_sh.at[pl.ds(s * _NPT + kk * 128, 128)])

    def zv(i, _):
        vout[i // 4, pl.ds((i % 4) * 16, 16)] = jnp.zeros((16,), jnp.float32)
        return _
    lax.fori_loop(0, _K1 * 4, zv, None)
    for kk in range(_NPT // _K1):
        pltpu.sync_copy(vout, acc_sh.at[pl.ds(s * _NPT + kk * _K1, _K1)])
    plsc.subcore_barrier()

    def idx_start(sl, i, ebase):
        base = ebase + i * _K1
        pltpu.async_copy(src_h.at[pl.ds(base, _K1)], idxsl[sl], semil[sl])
        pltpu.async_copy(dst_h.at[pl.ds(base, _K1)], idxdl[sl], semil[sl])

    def idx_wait(sl):
        pltpu.make_async_copy(src_h.at[pl.ds(0, _K1)], idxsl[sl], semil[sl]).wait()
        pltpu.make_async_copy(dst_h.at[pl.ds(0, _K1)], idxdl[sl], semil[sl]).wait()

    def p1_gather(sl, bb):
        pltpu.async_copy(ast_h.at[idxsl[sl]], ral[bb], seml[bb])
        pltpu.async_copy(adt_h.at[idxdl[sl]], rbl[bb], seml[bb])

    def p1_fin(sl, bb):
        pltpu.make_async_copy(ast_h.at[idxsl[sl]], ral[bb], seml[bb]).wait()
        pltpu.make_async_copy(adt_h.at[idxdl[sl]], rbl[bb], seml[bb]).wait()

        def cmp(e, _2):
            x = ral[bb][e, :] + rbl[bb][e, :]
            ral[bb][e, :] = jnp.exp(_leaky(x))
            return _2
        lax.fori_loop(0, _K1, cmp, None)
        pltpu.sync_copy(ral[bb], denom_sh.at[idxdl[sl]], add=True)

    eb1 = s * _P1T
    idx_start(0, 0, eb1)
    idx_start(1, 1, eb1)
    idx_wait(0)
    p1_gather(0, 0)

    def p1o(j, _):
        for u in range(6):
            i = 6 * j + u

            @pl.when(i + 2 < _N11)
            def _sti():
                idx_start((u + 2) % 3, i + 2, eb1)

            @pl.when(i + 1 < _N11)
            def _stg():
                idx_wait((u + 1) % 3)
                p1_gather((u + 1) % 3, (u + 1) % 2)
            p1_fin(u % 3, u % 2)
        return _
    lax.fori_loop(0, _N11 // 6, p1o, None)
    plsc.subcore_barrier()

    for kk in range(_NPT // 128):
        off = s * _NPT + kk * 128
        pltpu.sync_copy(denom_sh.at[pl.ds(off, 128)], strip)

        def rec(i, _):
            v = strip[i, :]
            strip[i, :] = 1.0 / (v + 1e-16)
            return _
        lax.fori_loop(0, 128, rec, None)
        pltpu.sync_copy(strip, recip_h.at[pl.ds(off, 128)])
    plsc.subcore_barrier()

    def p2_gather(sl, bb):
        pltpu.async_copy(ast_h.at[idxsl[sl]], ral[bb], seml[bb])
        pltpu.async_copy(adt_h.at[idxdl[sl]], rbl[bb], seml[bb])
        pltpu.async_copy(recip_h.at[idxdl[sl]], rrl[bb], seml[bb])
        pltpu.async_copy(ht_h.at[idxsl[sl]], hl[bb], seml[bb])

    def p2_fin(sl, bb):
        pltpu.make_async_copy(ast_h.at[idxsl[sl]], ral[bb], seml[bb]).wait()
        pltpu.make_async_copy(adt_h.at[idxdl[sl]], rbl[bb], seml[bb]).wait()
        pltpu.make_async_copy(recip_h.at[idxdl[sl]], rrl[bb], seml[bb]).wait()
        pltpu.make_async_copy(ht_h.at[idxsl[sl]], hl[bb], seml[bb]).wait()

        def cmp(e, _2):
            x = ral[bb][e, :] + rbl[bb][e, :]
            cf = jnp.exp(_leaky(x)) * rrl[bb][e, :]
            accs = [jnp.zeros((16,), jnp.float32) for _j in range(4)]
            for h in range(_HEADS):
                bc = _vbcast(cf, h)
                for j in range(4):
                    accs[j] = accs[j] + bc * hl[bb][e, pl.ds(h * 64 + j * 16, 16)]
            for j in range(4):
                vout[e, pl.ds(j * 16, 16)] = accs[j]
            return _2
        lax.fori_loop(0, _K1, cmp, None)
        pltpu.sync_copy(vout, acc_sh.at[idxdl[sl]], add=True)

    eb2 = c * (_EEP // 2) + s * _P2T
    idx_start(0, 0, eb2)
    idx_start(1, 1, eb2)
    idx_wait(0)
    p2_gather(0, 0)

    def p2o(j, _):
        for u in range(6):
            i = 6 * j + u

            @pl.when(i + 2 < _N21)
            def _sti():
                idx_start((u + 2) % 3, i + 2, eb2)

            @pl.when(i + 1 < _N21)
            def _stg():
                idx_wait((u + 1) % 3)
                p2_gather((u + 1) % 3, (u + 1) % 2)
            p2_fin(u % 3, u % 2)
        return _
    lax.fori_loop(0, _N21 // 6, p2o, None)
    plsc.subcore_barrier()

    for kk in range(_NPT // _K1):
        pltpu.sync_copy(acc_sh.at[pl.ds(s * _NPT + kk * _K1, _K1)],
                        part_h.at[pl.ds(c * _NP + s * _NPT + kk * _K1, _K1)])


def _sc_layer1(srcd, dstd, ht, ast, adt):
    mesh = plsc.VectorSubcoreMesh(core_axis_name="c", subcore_axis_name="s")
    f = pl.kernel(
        _l1_body,
        out_type=[jax.ShapeDtypeStruct((2 * _NP, _HID), jnp.float32),
                  jax.ShapeDtypeStruct((_NP, 16), jnp.float32)],
        mesh=mesh,
        compiler_params=pltpu.CompilerParams(use_tc_tiling_on_sc=False, needs_layout_passes=False),
        scratch_types=[
            pltpu.VMEM((_K1,), jnp.int32),
            pltpu.VMEM((_K1,), jnp.int32),
            pltpu.VMEM((_K1,), jnp.int32),
            pltpu.VMEM((_K1,), jnp.int32),
            pltpu.VMEM((_K1,), jnp.int32),
            pltpu.VMEM((_K1,), jnp.int32),
            pltpu.VMEM((_K1, 16), jnp.float32),
            pltpu.VMEM((_K1, 16), jnp.float32),
            pltpu.VMEM((_K1, 16), jnp.float32),
            pltpu.VMEM((_K1, 16), jnp.float32),
            pltpu.VMEM((_K1, 16), jnp.float32),
            pltpu.VMEM((_K1, 16), jnp.float32),
            pltpu.VMEM((_K1, 512), jnp.float32),
            pltpu.VMEM((_K1, 512), jnp.float32),
            pltpu.VMEM((_K1, _HID), jnp.float32),
            pltpu.VMEM((128, 16), jnp.float32),
            pltpu.VMEM_SHARED((_NP, 16), jnp.float32),
            pltpu.VMEM_SHARED((_NP, _HID), jnp.float32),
            pltpu.SemaphoreType.DMA,
            pltpu.SemaphoreType.DMA,
            pltpu.SemaphoreType.DMA,
            pltpu.SemaphoreType.DMA,
            pltpu.SemaphoreType.DMA,
        ],
    )
    return f(srcd, dstd, ht, ast, adt)


_K2 = 192
_N12 = _P1T // _K2
_N22 = _P2T // _K2


def _l2_body(src_h, dst_h, h2t_h, as2_h, ad2_h,
             part2_h,
             idxs0, idxd0, idxs1, idxd1, idxs2, idxd2, as2v, ad2v, rc2,
             exb0, exb1, cfb, hr0, hr1, vout2, d2chunk,
             h2_sh, denom2_sh, acc2_sh, sem0, sem1, semi0, semi1, semi2):
    c = lax.axis_index("c")
    s = lax.axis_index("s")
    idxsl, idxdl = (idxs0, idxs1, idxs2), (idxd0, idxd1, idxd2)
    hrl, seml = (hr0, hr1), (sem0, sem1)
    exbl = (exb0, exb1)
    semil = (semi0, semi1, semi2)

    pltpu.sync_copy(as2_h, as2v)
    pltpu.sync_copy(ad2_h, ad2v)
    pltpu.sync_copy(h2t_h.at[pl.ds(s * _NPT, _NPT)],
                    h2_sh.at[pl.ds(s * _NPT, _NPT)])

    def zd(i, _):
        d2chunk[pl.ds(i * 16, 16)] = jnp.zeros((16,), jnp.float32)
        return _
    lax.fori_loop(0, _NPT // 16, zd, None)
    pltpu.sync_copy(d2chunk, denom2_sh.at[pl.ds(s * _NPT, _NPT)])

    def zv(i, _):
        vout2[i // 3, pl.ds((i % 3) * 16, 16)] = jnp.zeros((16,), jnp.float32)
        return _
    lax.fori_loop(0, _K2 * 3, zv, None)
    for kk in range(0, _NPT, _K2):
        nrow = min(_K2, _NPT - kk)
        pltpu.sync_copy(vout2.at[pl.ds(0, nrow)],
                        acc2_sh.at[pl.ds(s * _NPT + kk, nrow)])
    plsc.subcore_barrier()

    def idx_start(sl, i, ebase):
        base = ebase + i * _K2
        pltpu.async_copy(src_h.at[pl.ds(base, _K2)], idxsl[sl], semil[sl])
        pltpu.async_copy(dst_h.at[pl.ds(base, _K2)], idxdl[sl], semil[sl])

    def idx_wait(sl):
        pltpu.make_async_copy(src_h.at[pl.ds(0, _K2)], idxsl[sl], semil[sl]).wait()
        pltpu.make_async_copy(dst_h.at[pl.ds(0, _K2)], idxdl[sl], semil[sl]).wait()

    def p1_fin(sl, bb):
        def cmp(t, _2):
            sv = idxsl[sl][pl.ds(t * 16, 16)]
            dv = idxdl[sl][pl.ds(t * 16, 16)]
            aa = plsc.load_gather(as2v, [sv])
            ab = plsc.load_gather(ad2v, [dv])
            exbl[bb][pl.ds(t * 16, 16)] = jnp.exp(_leaky(aa + ab))
            return _2
        lax.fori_loop(0, _K2 // 16, cmp, None)
        pltpu.sync_copy(exbl[bb], denom2_sh.at[idxdl[sl]], add=True)

    eb1 = s * _P1T
    idx_start(0, 0, eb1)
    idx_start(1, 1, eb1)
    idx_wait(0)

    def p1o(j, _):
        for u in range(6):
            i = 6 * j + u

            @pl.when(i + 2 < _N12)
            def _sti():
                idx_start((u + 2) % 3, i + 2, eb1)

            @pl.when(i + 1 < _N12)
            def _stw():
                idx_wait((u + 1) % 3)
            p1_fin(u % 3, u % 2)
        return _
    lax.fori_loop(0, _N12 // 6, p1o, None)
    plsc.subcore_barrier()

    pltpu.sync_copy(denom2_sh.at[pl.ds(s * _NPT, _NPT)], d2chunk)

    def rec(i, _):
        v = d2chunk[pl.ds(i * 16, 16)]
        d2chunk[pl.ds(i * 16, 16)] = 1.0 / (v + 1e-16)
        return _
    lax.fori_loop(0, _NPT // 16, rec, None)
    pltpu.sync_copy(d2chunk, denom2_sh.at[pl.ds(s * _NPT, _NPT)])
    plsc.subcore_barrier()
    pltpu.sync_copy(denom2_sh, rc2)

    def p2_gather(sl, bb):
        pltpu.async_copy(h2_sh.at[idxsl[sl]], hrl[bb], seml[bb])

    def p2_fin(sl, bb):
        pltpu.make_async_copy(h2_sh.at[idxsl[sl]], hrl[bb], seml[bb]).wait()

        def cmp(t, _2):
            sv = idxsl[sl][pl.ds(t * 16, 16)]
            dv = idxdl[sl][pl.ds(t * 16, 16)]
            aa = plsc.load_gather(as2v, [sv])
            ab = plsc.load_gather(ad2v, [dv])
            r = plsc.load_gather(rc2, [dv])
            cfb[pl.ds(t * 16, 16)] = jnp.exp(_leaky(aa + ab)) * r
            return _2
        lax.fori_loop(0, _K2 // 16, cmp, None)

        def rowm(e, _2):
            bc = _vbcast(cfb[pl.ds((e // 16) * 16, 16)], e % 16)
            for j in range(3):
                vout2[e, pl.ds(j * 16, 16)] = bc * hrl[bb][e, pl.ds(j * 16, 16)]
            return _2
        lax.fori_loop(0, _K2, rowm, None)
        pltpu.sync_copy(vout2, acc2_sh.at[idxdl[sl]], add=True)

    eb2 = c * (_EEP // 2) + s * _P2T
    idx_start(0, 0, eb2)
    idx_start(1, 1, eb2)
    idx_wait(0)
    p2_gather(0, 0)

    def p2o(j, _):
        for u in range(6):
            i = 6 * j + u

            @pl.when(i + 2 < _N22)
            def _sti():
                idx_start((u + 2) % 3, i + 2, eb2)

            @pl.when(i + 1 < _N22)
            def _stg():
                idx_wait((u + 1) % 3)
                p2_gather((u + 1) % 3, (u + 1) % 2)
            p2_fin(u % 3, u % 2)
        return _
    lax.fori_loop(0, _N22 // 6, p2o, None)
    plsc.subcore_barrier()

    for kk in range(0, _NPT, _K2):
        nrow = min(_K2, _NPT - kk)
        pltpu.sync_copy(acc2_sh.at[pl.ds(s * _NPT + kk, nrow)],
                        part2_h.at[pl.ds(c * _NP + s * _NPT + kk, nrow)])


def _sc_layer2(srcd, dstd, h2t, as2v, ad2v):
    mesh = plsc.VectorSubcoreMesh(core_axis_name="c", subcore_axis_name="s")
    f = pl.kernel(
        _l2_body,
        out_type=jax.ShapeDtypeStruct((2 * _NP, _C2), jnp.float32),
        mesh=mesh,
        compiler_params=pltpu.CompilerParams(use_tc_tiling_on_sc=False, needs_layout_passes=False),
        scratch_types=[
            pltpu.VMEM((_K2,), jnp.int32),
            pltpu.VMEM((_K2,), jnp.int32),
            pltpu.VMEM((_K2,), jnp.int32),
            pltpu.VMEM((_K2,), jnp.int32),
            pltpu.VMEM((_K2,), jnp.int32),
            pltpu.VMEM((_K2,), jnp.int32),
            pltpu.VMEM((_NP,), jnp.float32),
            pltpu.VMEM((_NP,), jnp.float32),
            pltpu.VMEM((_NP,), jnp.float32),
            pltpu.VMEM((_K2,), jnp.float32),
            pltpu.VMEM((_K2,), jnp.float32),
            pltpu.VMEM((_K2,), jnp.float32),
            pltpu.VMEM((_K2, _C2), jnp.float32),
            pltpu.VMEM((_K2, _C2), jnp.float32),
            pltpu.VMEM((_K2, _C2), jnp.float32),
            pltpu.VMEM((_NPT,), jnp.float32),
            pltpu.VMEM_SHARED((_NP, _C2), jnp.float32),
            pltpu.VMEM_SHARED((_NP,), jnp.float32),
            pltpu.VMEM_SHARED((_NP, _C2), jnp.float32),
            pltpu.SemaphoreType.DMA,
            pltpu.SemaphoreType.DMA,
            pltpu.SemaphoreType.DMA,
            pltpu.SemaphoreType.DMA,
            pltpu.SemaphoreType.DMA,
        ],
    )
    return f(srcd, dstd, h2t, as2v, ad2v)


def kernel(x, edge_index, W1, a_src1, a_dst1, b1, W2, a_src2, a_dst2, b2):
    sl = jnp.arange(_N, dtype=jnp.int32)
    npad = _EEP - _EE
    padidx = _N + (jnp.arange(npad, dtype=jnp.int32) % 240)
    srcd = jnp.concatenate([edge_index[0].astype(jnp.int32), sl, padidx])
    dstd = jnp.concatenate([edge_index[1].astype(jnp.int32), sl, padidx])

    rows = jnp.arange(512, dtype=jnp.int32)
    hcol = rows // _HID
    asw = jnp.zeros((512, 16), jnp.float32).at[rows, hcol].set(
        a_src1.reshape(512))
    adw = jnp.zeros((512, 16), jnp.float32).at[rows, hcol].set(
        a_dst1.reshape(512))

    xp = jnp.pad(x, ((0, _NP - _N), (0, 0)))
    ht, ast, adt = _mm1(xp, W1, asw, adw)

    part, _recip = _sc_layer1(srcd, dstd, ht, ast, adt)

    w_as2 = (W2 @ a_src2[0]).reshape(_HID, 1)
    w_ad2 = (W2 @ a_dst2[0]).reshape(_HID, 1)
    w2e = jnp.concatenate(
        [W2, w_as2, w_ad2, jnp.zeros((_HID, _C2 - _NCLS - 2), jnp.float32)],
        axis=1)
    b1r = b1.reshape(1, _HID)
    h2t = _mid(part, b1r, w2e)
    as2v = h2t[:, _NCLS]
    ad2v = h2t[:, _NCLS + 1]

    part2 = _sc_layer2(srcd, dstd, h2t, as2v, ad2v)

    b2r = jnp.pad(b2, (0, _C2 - _NCLS)).reshape(1, _C2)
    o = _fin(part2, b2r)
    return o[:_N, :_NCLS]
